# baseline (speedup 1.0000x reference)
import functools

import jax
import jax.numpy as jnp
from jax import lax
from jax.experimental import pallas as pl
from jax.experimental.pallas import tpu as pltpu
from jax.experimental.pallas import tpu_sc as plsc

_N = 10000
_NP = 10240
_NH = 5120
_E = 320000
_DIN = 128
_DH = 128
_DOUT = 64
_K = 10

_NC = 2
_NS = 16
_NW = _NC * _NS
_EPW = _E // _NW

_W = 5520
_DUMC = 5500
_ACCR = 5760
_ZSTR = _ACCR // _NS
_CSTR = _NH // _NS
_SUB = 4
_CSUB = _CSTR // _SUB

_C = 120
_CH = 2 * _W // _C
_CHP = _CH + 2

_HD = 16
_HC = 125
_HCH = _EPW // _HC
_HRPT = _NP // _NS
_ZR = 128

_mesh = plsc.VectorSubcoreMesh(core_axis_name="c", subcore_axis_name="s")
_sc_params = pltpu.CompilerParams(use_tc_tiling_on_sc=False)
_sc_params_nlp = pltpu.CompilerParams(use_tc_tiling_on_sc=False,
                                      needs_layout_passes=False)


def _fill_vmem_2d(ref, rows, cols, vec):
    def body(i, carry):
        for c in range(cols // 16):
            ref[i, pl.ds(c * 16, 16)] = vec
        return carry

    lax.fori_loop(0, rows, body, 0)


@functools.partial(
    pl.kernel,
    mesh=_mesh,
    out_type=jax.ShapeDtypeStruct((_NC, _NP, _HD), jnp.float32),
    scratch_types=[
        pltpu.VMEM((_HCH, _HC), jnp.int32),
        pltpu.VMEM((_HC, _HD), jnp.float32),
        pltpu.VMEM((_ZR, _HD), jnp.float32),
        pltpu.VMEM_SHARED((_NP, _HD), jnp.float32),
    ],
    compiler_params=_sc_params,
)
def _hist_sc(col_hbm, out_hbm, colv, onesb, zbuf, acc):
    cid = lax.axis_index("c")
    sid = lax.axis_index("s")
    wid = cid * _NS + sid

    _fill_vmem_2d(zbuf, _ZR, _HD, jnp.zeros((16,), jnp.float32))
    _fill_vmem_2d(onesb, _HC, _HD, jnp.ones((16,), jnp.float32))

    for z in range(_HRPT // _ZR):
        pltpu.sync_copy(zbuf, acc.at[pl.ds(sid * _HRPT + z * _ZR, _ZR)])
    plsc.subcore_barrier()

    pltpu.sync_copy(col_hbm.at[wid], colv)

    def chunk(j, carry):
        pltpu.sync_copy(onesb, acc.at[colv.at[j]], add=True)
        return carry

    lax.fori_loop(0, _HCH, chunk, 0)

    plsc.subcore_barrier()
    pltpu.sync_copy(
        acc.at[pl.ds(sid * _HRPT, _HRPT)],
        out_hbm.at[cid, pl.ds(sid * _HRPT, _HRPT)],
    )


@functools.partial(
    pl.kernel,
    mesh=_mesh,
    out_type=[jax.ShapeDtypeStruct((_NW, _W), jnp.int32) for _ in range(4)],
    scratch_types=[
        pltpu.VMEM((_EPW,), jnp.int32),
        pltpu.VMEM((_EPW,), jnp.int32),
        pltpu.VMEM((_W,), jnp.int32),
        pltpu.VMEM((_W,), jnp.int32),
        pltpu.VMEM((_W,), jnp.int32),
        pltpu.VMEM((_W,), jnp.int32),
        pltpu.VMEM((32,), jnp.int32),
    ],
    compiler_params=_sc_params_nlp,
)
def _part_sc(row_hbm, col_hbm, ar_hbm, ac_hbm, br_hbm, bc_hbm,
             rowf, colf, lar, lac, lbr, lbc, offs):
    cid = lax.axis_index("c")
    sid = lax.axis_index("s")
    wid = cid * _NS + sid

    pltpu.sync_copy(row_hbm.at[wid], rowf)
    pltpu.sync_copy(col_hbm.at[wid], colf)

    zed = jnp.zeros((16,), jnp.int32)
    dum = jnp.full((16,), _DUMC, jnp.int32)

    def prefill(i, carry):
        lar[pl.ds(i * 16, 16)] = zed
        lac[pl.ds(i * 16, 16)] = dum
        lbr[pl.ds(i * 16, 16)] = zed
        lbc[pl.ds(i * 16, 16)] = dum
        return carry

    lax.fori_loop(0, _W // 16, prefill, 0)

    offs[pl.ds(0, 16)] = jnp.zeros((16,), jnp.int32)
    offs[pl.ds(16, 16)] = jnp.zeros((16,), jnp.int32)
    lane1 = lax.iota(jnp.int32, 16) + 1

    def step(k, carry):
        rv = rowf[pl.ds(k * 16, 16)]
        cv = colf[pl.ds(k * 16, 16)]
        ma = cv < _NH
        mb = jnp.logical_not(ma)
        mai = jnp.where(ma, 1, 0)
        incla = plsc.cumsum(mai)
        inclb = lane1 - incla
        offa = offs[pl.ds(0, 16)]
        offb = offs[pl.ds(16, 16)]
        plsc.store_scatter(lar, [offa + incla - mai], rv, mask=ma)
        plsc.store_scatter(lac, [offa + incla - mai], cv, mask=ma)
        exclb = inclb - jnp.where(mb, 1, 0)
        plsc.store_scatter(lbr, [offb + exclb], rv, mask=mb)
        plsc.store_scatter(lbc, [offb + exclb], cv - _NH, mask=mb)
        tota = plsc.cummax(lax.rev(incla, (0,)))
        offs[pl.ds(0, 16)] = jnp.minimum(offa + tota, _W - 16)
        offs[pl.ds(16, 16)] = jnp.minimum(offb + 16 - tota, _W - 16)
        return carry

    lax.fori_loop(0, _EPW // 16, step, 0)

    pltpu.sync_copy(lar, ar_hbm.at[wid])
    pltpu.sync_copy(lac, ac_hbm.at[wid])
    pltpu.sync_copy(lbr, br_hbm.at[wid])
    pltpu.sync_copy(lbc, bc_hbm.at[wid])


@functools.partial(
    pl.kernel,
    mesh=_mesh,
    out_type=jax.ShapeDtypeStruct((_NP, _DOUT), jnp.float32),
    scratch_types=[
        pltpu.VMEM((_CHP, _C), jnp.int32),
        pltpu.VMEM((_CHP, _C), jnp.int32),
        pltpu.VMEM((_C, _DOUT), jnp.float32),
        pltpu.VMEM((_C, _DOUT), jnp.float32),
        pltpu.VMEM((_C, _DOUT), jnp.float32),
        pltpu.VMEM((_C, _DOUT), jnp.float32),
        pltpu.VMEM((_C, _DOUT), jnp.float32),
        pltpu.VMEM((_CSUB, _DOUT), jnp.float32),
        pltpu.VMEM((_CSUB, _DOUT), jnp.float32),
        pltpu.VMEM((_CSUB, _DOUT), jnp.float32),
        pltpu.VMEM_SHARED((_ACCR, _DOUT), jnp.float32),
        [pltpu.SemaphoreType.DMA] * 4,
        [pltpu.SemaphoreType.DMA] * 4,
        pltpu.SemaphoreType.DMA,
    ],
    compiler_params=_sc_params,
)
def _hop_sc(g_hbm, row_hbm, col_hbm, d2_hbm, gout_hbm, rowv, colv,
            b0, b1, b2, b3, zbuf, ab, gb, db, acc, gsem, ssem, zsem):
    cid = lax.axis_index("c")
    sid = lax.axis_index("s")
    bufs = (b0, b1, b2, b3)

    def g_start(j, slot):
        pltpu.async_copy(g_hbm.at[rowv.at[j]], bufs[slot], gsem[slot])

    def g_wait(slot):
        pltpu.make_async_copy(g_hbm.at[rowv.at[0]], bufs[slot], gsem[slot]).wait()

    def s_start(j, slot):
        pltpu.async_copy(bufs[slot], acc.at[colv.at[j]], ssem[slot], add=True)

    def s_wait(slot):
        pltpu.make_async_copy(bufs[slot], acc.at[colv.at[0]], ssem[slot]).wait()

    _fill_vmem_2d(zbuf, _C, _DOUT, jnp.zeros((16,), jnp.float32))
    for z in range(_ZSTR // _C):
        pltpu.async_copy(zbuf, acc.at[pl.ds(sid * _ZSTR + z * _C, _C)], zsem)
    pltpu.sync_copy(row_hbm.at[cid, sid], rowv)
    pltpu.sync_copy(col_hbm.at[cid, sid], colv)
    for z in range(_ZSTR // _C):
        pltpu.make_async_copy(zbuf, acc.at[pl.ds(sid * _ZSTR, _C)], zsem).wait()
    plsc.subcore_barrier()

    g_start(0, 0)
    g_start(1, 1)
    g_wait(0); s_start(0, 0); g_start(2, 2)
    g_wait(1); s_start(1, 1); g_start(3, 3)
    g_wait(2); s_start(2, 2); s_wait(0); g_start(4, 0)
    g_wait(3); s_start(3, 3); s_wait(1); g_start(5, 1)

    def group(gi, carry):
        base = 4 * gi
        for b in range(4):
            j = base + b
            g_wait(b)
            s_start(j, b)
            s_wait((b + 2) % 4)
            g_start(j + 2, (b + 2) % 4)
        return carry

    lax.fori_loop(1, _CH // 4, group, 0)
    g_wait(0)
    g_wait(1)
    s_wait(2)
    s_wait(3)

    plsc.subcore_barrier()

    for sub in range(_SUB):
        loc = sid * _CSTR + sub * _CSUB
        gbase = cid * _NH + sid * _CSTR + sub * _CSUB
        pltpu.sync_copy(acc.at[pl.ds(loc, _CSUB)], ab)
        pltpu.sync_copy(g_hbm.at[pl.ds(gbase, _CSUB)], gb)
        pltpu.sync_copy(d2_hbm.at[pl.ds(gbase, _CSUB)], db)

        def crow(i, carry):
            for q in range(_DOUT // 16):
                sl = pl.ds(q * 16, 16)
                ab[i, sl] = db[i, sl] * (ab[i, sl] + gb[i, sl])
            return carry

        lax.fori_loop(0, _CSUB, crow, 0)
        pltpu.sync_copy(ab, gout_hbm.at[pl.ds(gbase, _CSUB)])


def _mlp_body(x_ref, w1_ref, b1_ref, w2_ref, b2_ref, o_ref):
    h = jnp.dot(x_ref[...], w1_ref[...], preferred_element_type=jnp.float32)
    h = jnp.maximum(h + b1_ref[...], 0.0)
    o_ref[...] = (
        jnp.dot(h, w2_ref[...], preferred_element_type=jnp.float32) + b2_ref[...]
    )


_MLP_R = 1024


def _mlp(x, w1, b1, w2, b2):
    return pl.pallas_call(
        _mlp_body,
        grid=(_NP // _MLP_R,),
        in_specs=[
            pl.BlockSpec((_MLP_R, _DIN), lambda i: (i, 0)),
            pl.BlockSpec((_DIN, _DH), lambda i: (0, 0)),
            pl.BlockSpec((1, _DH), lambda i: (0, 0)),
            pl.BlockSpec((_DH, _DOUT), lambda i: (0, 0)),
            pl.BlockSpec((1, _DOUT), lambda i: (0, 0)),
        ],
        out_specs=pl.BlockSpec((_MLP_R, _DOUT), lambda i: (i, 0)),
        out_shape=jax.ShapeDtypeStruct((_NP, _DOUT), jnp.float32),
    )(x, w1, b1.reshape(1, _DH), w2, b2.reshape(1, _DOUT))


def _prep_body(dp_ref, h_ref, g_ref, d2_ref, sd_ref):
    deg = dp_ref[0] + dp_ref[1] + 1.0
    r = deg.shape[0]
    dinv = lax.rsqrt(deg)[:, 0:1]
    dinvb = jnp.broadcast_to(dinv, (r, _DOUT))
    g_ref[...] = dinvb * h_ref[...]
    d2_ref[...] = dinvb * dinvb
    sd_ref[...] = jnp.broadcast_to(jnp.sqrt(deg)[:, 0:1], (r, _DOUT))


_EW_R = 1024


def _prep(dp, h0):
    return pl.pallas_call(
        _prep_body,
        grid=(_NP // _EW_R,),
        in_specs=[
            pl.BlockSpec((_NC, _EW_R, _HD), lambda i: (0, i, 0)),
            pl.BlockSpec((_EW_R, _DOUT), lambda i: (i, 0)),
        ],
        out_specs=[
            pl.BlockSpec((_EW_R, _DOUT), lambda i: (i, 0)),
            pl.BlockSpec((_EW_R, _DOUT), lambda i: (i, 0)),
            pl.BlockSpec((_EW_R, _DOUT), lambda i: (i, 0)),
        ],
        out_shape=[
            jax.ShapeDtypeStruct((_NP, _DOUT), jnp.float32),
            jax.ShapeDtypeStruct((_NP, _DOUT), jnp.float32),
            jax.ShapeDtypeStruct((_NP, _DOUT), jnp.float32),
        ],
    )(dp, h0)


def _final_body(h_ref, sd_ref, t_ref, *refs):
    g_refs, hid_ref = refs[:_K], refs[_K]
    s = t_ref[0, 1] * g_refs[0][...]
    for k in range(1, _K):
        s = s + t_ref[0, k + 1] * g_refs[k][...]
    hid_ref[...] = t_ref[0, 0] * h_ref[...] + sd_ref[...] * s


def _final(h0, sd, tvec, gs):
    return pl.pallas_call(
        _final_body,
        grid=(_NP // _EW_R,),
        in_specs=[
            pl.BlockSpec((_EW_R, _DOUT), lambda i: (i, 0)),
            pl.BlockSpec((_EW_R, _DOUT), lambda i: (i, 0)),
            pl.BlockSpec((1, _K + 1), lambda i: (0, 0)),
        ] + [pl.BlockSpec((_EW_R, _DOUT), lambda i: (i, 0)) for _ in range(_K)],
        out_specs=pl.BlockSpec((_EW_R, _DOUT), lambda i: (i, 0)),
        out_shape=jax.ShapeDtypeStruct((_NP, _DOUT), jnp.float32),
    )(h0, sd, tvec, *gs)


def kernel(x, edge_index, W1, b1, W2, b2, temp):
    row2 = edge_index[0].reshape(_NW, _EPW)
    col2 = edge_index[1].reshape(_NW, _EPW)

    xp = jnp.pad(x, ((0, _NP - _N), (0, 0)))
    h0 = _mlp(xp, W1, b1, W2, b2)
    dp = _hist_sc(col2.reshape(_NW, _HCH, _HC))
    g, d2, sd = _prep(dp, h0)

    ar, ac, br, bc = _part_sc(row2, col2)

    def slabs(a, b):
        s = jnp.stack([a, b]).reshape(_NC, _NS, _CH, _C)
        return jnp.concatenate([s, s[:, :, :2]], axis=2)

    rows = slabs(ar, br)
    cols = slabs(ac, bc)

    gs = []
    for _ in range(_K):
        g = _hop_sc(g, rows, cols, d2)
        gs.append(g)

    hidden = _final(h0, sd, temp.reshape(1, _K + 1), gs)
    return hidden[:_N]


# speedup vs baseline: 7.7241x; 7.7241x over previous
import functools

import jax
import jax.numpy as jnp
from jax import lax
from jax.experimental import pallas as pl
from jax.experimental.pallas import tpu as pltpu
from jax.experimental.pallas import tpu_sc as plsc

_N = 10000
_NP = 10240
_NH = 5120
_E = 320000
_DIN = 128
_DH = 128
_DOUT = 64
_K = 10

_NC = 2
_NS = 16
_NW = _NC * _NS
_EPW = _E // _NW

_W = 5520
_DUMC = 5500
_ACCR = 5760
_ZSTR = _ACCR // _NS
_CSTR = _NH // _NS
_SUB = 4
_CSUB = _CSTR // _SUB

_C = 120
_CH = 2 * _W // _C
_CHP = _CH + 2

_HD = 16
_HC = 125
_HCH = _EPW // _HC
_HRPT = _NP // _NS
_ZR = 128

_mesh = plsc.VectorSubcoreMesh(core_axis_name="c", subcore_axis_name="s")
_sc_params = pltpu.CompilerParams(use_tc_tiling_on_sc=False)
_sc_params_nlp = pltpu.CompilerParams(use_tc_tiling_on_sc=False,
                                      needs_layout_passes=False)


def _fill_vmem_2d(ref, rows, cols, vec):
    def body(i, carry):
        for c in range(cols // 16):
            ref[i, pl.ds(c * 16, 16)] = vec
        return carry

    lax.fori_loop(0, rows, body, 0)


@functools.partial(
    pl.kernel,
    mesh=_mesh,
    out_type=jax.ShapeDtypeStruct((_NC, _NP, _HD), jnp.float32),
    scratch_types=[
        pltpu.VMEM((_HCH, _HC), jnp.int32),
        pltpu.VMEM((_HC, _HD), jnp.float32),
        pltpu.VMEM((_ZR, _HD), jnp.float32),
        pltpu.VMEM_SHARED((_NP, _HD), jnp.float32),
    ],
    compiler_params=_sc_params,
)
def _hist_sc(col_hbm, out_hbm, colv, onesb, zbuf, acc):
    cid = lax.axis_index("c")
    sid = lax.axis_index("s")
    wid = cid * _NS + sid

    _fill_vmem_2d(zbuf, _ZR, _HD, jnp.zeros((16,), jnp.float32))
    _fill_vmem_2d(onesb, _HC, _HD, jnp.ones((16,), jnp.float32))

    for z in range(_HRPT // _ZR):
        pltpu.sync_copy(zbuf, acc.at[pl.ds(sid * _HRPT + z * _ZR, _ZR)])
    plsc.subcore_barrier()

    pltpu.sync_copy(col_hbm.at[wid], colv)

    def chunk(j, carry):
        pltpu.sync_copy(onesb, acc.at[colv.at[j]], add=True)
        return carry

    lax.fori_loop(0, _HCH, chunk, 0)

    plsc.subcore_barrier()
    pltpu.sync_copy(
        acc.at[pl.ds(sid * _HRPT, _HRPT)],
        out_hbm.at[cid, pl.ds(sid * _HRPT, _HRPT)],
    )


@functools.partial(
    pl.kernel,
    mesh=_mesh,
    out_type=[jax.ShapeDtypeStruct((_NW, _W), jnp.int32) for _ in range(4)],
    scratch_types=[
        pltpu.VMEM((_EPW,), jnp.int32),
        pltpu.VMEM((_EPW,), jnp.int32),
        pltpu.VMEM((_W,), jnp.int32),
        pltpu.VMEM((_W,), jnp.int32),
        pltpu.VMEM((_W,), jnp.int32),
        pltpu.VMEM((_W,), jnp.int32),
        pltpu.VMEM((32,), jnp.int32),
    ],
    compiler_params=_sc_params_nlp,
)
def _part_sc(row_hbm, col_hbm, ar_hbm, ac_hbm, br_hbm, bc_hbm,
             rowf, colf, lar, lac, lbr, lbc, offs):
    cid = lax.axis_index("c")
    sid = lax.axis_index("s")
    wid = cid * _NS + sid

    pltpu.sync_copy(row_hbm.at[wid], rowf)
    pltpu.sync_copy(col_hbm.at[wid], colf)

    lane = lax.iota(jnp.int32, 16)

    def prefill(i, carry):
        spread = i * 16 + lane
        lar[pl.ds(i * 16, 16)] = spread & 1023
        lac[pl.ds(i * 16, 16)] = _NH + (spread & 511)
        lbr[pl.ds(i * 16, 16)] = spread & 1023
        lbc[pl.ds(i * 16, 16)] = _NH + (spread & 511)
        return carry

    lax.fori_loop(0, _W // 16, prefill, 0)

    offs[pl.ds(0, 16)] = jnp.zeros((16,), jnp.int32)
    offs[pl.ds(16, 16)] = jnp.zeros((16,), jnp.int32)
    lane1 = lax.iota(jnp.int32, 16) + 1

    def step(k, carry):
        rv = rowf[pl.ds(k * 16, 16)]
        cv = colf[pl.ds(k * 16, 16)]
        ma = cv < _NH
        mb = jnp.logical_not(ma)
        mai = jnp.where(ma, 1, 0)
        incla = plsc.cumsum(mai)
        inclb = lane1 - incla
        offa = offs[pl.ds(0, 16)]
        offb = offs[pl.ds(16, 16)]
        plsc.store_scatter(lar, [offa + incla - mai], rv, mask=ma)
        plsc.store_scatter(lac, [offa + incla - mai], cv, mask=ma)
        exclb = inclb - jnp.where(mb, 1, 0)
        plsc.store_scatter(lbr, [offb + exclb], rv, mask=mb)
        plsc.store_scatter(lbc, [offb + exclb], cv - _NH, mask=mb)
        tota = plsc.cummax(lax.rev(incla, (0,)))
        offs[pl.ds(0, 16)] = jnp.minimum(offa + tota, _W - 16)
        offs[pl.ds(16, 16)] = jnp.minimum(offb + 16 - tota, _W - 16)
        return carry

    lax.fori_loop(0, _EPW // 16, step, 0)

    pltpu.sync_copy(lar, ar_hbm.at[wid])
    pltpu.sync_copy(lac, ac_hbm.at[wid])
    pltpu.sync_copy(lbr, br_hbm.at[wid])
    pltpu.sync_copy(lbc, bc_hbm.at[wid])


@functools.partial(
    pl.kernel,
    mesh=_mesh,
    out_type=jax.ShapeDtypeStruct((_NP, _DOUT), jnp.float32),
    scratch_types=[
        pltpu.VMEM((_CHP, _C), jnp.int32),
        pltpu.VMEM((_CHP, _C), jnp.int32),
        pltpu.VMEM((_C, _DOUT), jnp.float32),
        pltpu.VMEM((_C, _DOUT), jnp.float32),
        pltpu.VMEM((_C, _DOUT), jnp.float32),
        pltpu.VMEM((_C, _DOUT), jnp.float32),
        pltpu.VMEM((_C, _DOUT), jnp.float32),
        pltpu.VMEM((_CSUB, _DOUT), jnp.float32),
        pltpu.VMEM((_CSUB, _DOUT), jnp.float32),
        pltpu.VMEM((_CSUB, _DOUT), jnp.float32),
        pltpu.VMEM_SHARED((_ACCR, _DOUT), jnp.float32),
        [pltpu.SemaphoreType.DMA] * 4,
        [pltpu.SemaphoreType.DMA] * 4,
        pltpu.SemaphoreType.DMA,
    ],
    compiler_params=_sc_params,
)
def _hop_sc(g_hbm, row_hbm, col_hbm, d2_hbm, gout_hbm, rowv, colv,
            b0, b1, b2, b3, zbuf, ab, gb, db, acc, gsem, ssem, zsem):
    cid = lax.axis_index("c")
    sid = lax.axis_index("s")
    bufs = (b0, b1, b2, b3)

    def g_start(j, slot):
        pltpu.async_copy(g_hbm.at[rowv.at[j]], bufs[slot], gsem[slot])

    def g_wait(slot):
        pltpu.make_async_copy(g_hbm.at[rowv.at[0]], bufs[slot], gsem[slot]).wait()

    def s_start(j, slot):
        pltpu.async_copy(bufs[slot], acc.at[colv.at[j]], ssem[slot], add=True)

    def s_wait(slot):
        pltpu.make_async_copy(bufs[slot], acc.at[colv.at[0]], ssem[slot]).wait()

    _fill_vmem_2d(zbuf, _C, _DOUT, jnp.zeros((16,), jnp.float32))
    for z in range(_ZSTR // _C):
        pltpu.async_copy(zbuf, acc.at[pl.ds(sid * _ZSTR + z * _C, _C)], zsem)
    pltpu.sync_copy(row_hbm.at[cid, sid], rowv)
    pltpu.sync_copy(col_hbm.at[cid, sid], colv)
    for z in range(_ZSTR // _C):
        pltpu.make_async_copy(zbuf, acc.at[pl.ds(sid * _ZSTR, _C)], zsem).wait()
    plsc.subcore_barrier()

    g_start(0, 0)
    g_start(1, 1)
    g_wait(0); s_start(0, 0); g_start(2, 2)
    g_wait(1); s_start(1, 1); g_start(3, 3)
    g_wait(2); s_start(2, 2); s_wait(0); g_start(4, 0)
    g_wait(3); s_start(3, 3); s_wait(1); g_start(5, 1)

    def group(gi, carry):
        base = 4 * gi
        for b in range(4):
            j = base + b
            g_wait(b)
            s_start(j, b)
            s_wait((b + 2) % 4)
            g_start(j + 2, (b + 2) % 4)
        return carry

    lax.fori_loop(1, _CH // 4, group, 0)
    g_wait(0)
    g_wait(1)
    s_wait(2)
    s_wait(3)

    plsc.subcore_barrier()

    for sub in range(_SUB):
        loc = sid * _CSTR + sub * _CSUB
        gbase = cid * _NH + sid * _CSTR + sub * _CSUB
        pltpu.sync_copy(acc.at[pl.ds(loc, _CSUB)], ab)
        pltpu.sync_copy(g_hbm.at[pl.ds(gbase, _CSUB)], gb)
        pltpu.sync_copy(d2_hbm.at[pl.ds(gbase, _CSUB)], db)

        def crow(i, carry):
            for q in range(_DOUT // 16):
                sl = pl.ds(q * 16, 16)
                ab[i, sl] = db[i, sl] * (ab[i, sl] + gb[i, sl])
            return carry

        lax.fori_loop(0, _CSUB, crow, 0)
        pltpu.sync_copy(ab, gout_hbm.at[pl.ds(gbase, _CSUB)])


def _mlp_body(x_ref, w1_ref, b1_ref, w2_ref, b2_ref, o_ref):
    h = jnp.dot(x_ref[...], w1_ref[...], preferred_element_type=jnp.float32)
    h = jnp.maximum(h + b1_ref[...], 0.0)
    o_ref[...] = (
        jnp.dot(h, w2_ref[...], preferred_element_type=jnp.float32) + b2_ref[...]
    )


_MLP_R = 1024


def _mlp(x, w1, b1, w2, b2):
    return pl.pallas_call(
        _mlp_body,
        grid=(_NP // _MLP_R,),
        in_specs=[
            pl.BlockSpec((_MLP_R, _DIN), lambda i: (i, 0)),
            pl.BlockSpec((_DIN, _DH), lambda i: (0, 0)),
            pl.BlockSpec((1, _DH), lambda i: (0, 0)),
            pl.BlockSpec((_DH, _DOUT), lambda i: (0, 0)),
            pl.BlockSpec((1, _DOUT), lambda i: (0, 0)),
        ],
        out_specs=pl.BlockSpec((_MLP_R, _DOUT), lambda i: (i, 0)),
        out_shape=jax.ShapeDtypeStruct((_NP, _DOUT), jnp.float32),
    )(x, w1, b1.reshape(1, _DH), w2, b2.reshape(1, _DOUT))


def _prep_body(dp_ref, h_ref, g_ref, d2_ref, sd_ref):
    deg = dp_ref[0] + dp_ref[1] + 1.0
    r = deg.shape[0]
    dinv = lax.rsqrt(deg)[:, 0:1]
    dinvb = jnp.broadcast_to(dinv, (r, _DOUT))
    g_ref[...] = dinvb * h_ref[...]
    d2_ref[...] = dinvb * dinvb
    sd_ref[...] = jnp.broadcast_to(jnp.sqrt(deg)[:, 0:1], (r, _DOUT))


_EW_R = 1024


def _prep(dp, h0):
    return pl.pallas_call(
        _prep_body,
        grid=(_NP // _EW_R,),
        in_specs=[
            pl.BlockSpec((_NC, _EW_R, _HD), lambda i: (0, i, 0)),
            pl.BlockSpec((_EW_R, _DOUT), lambda i: (i, 0)),
        ],
        out_specs=[
            pl.BlockSpec((_EW_R, _DOUT), lambda i: (i, 0)),
            pl.BlockSpec((_EW_R, _DOUT), lambda i: (i, 0)),
            pl.BlockSpec((_EW_R, _DOUT), lambda i: (i, 0)),
        ],
        out_shape=[
            jax.ShapeDtypeStruct((_NP, _DOUT), jnp.float32),
            jax.ShapeDtypeStruct((_NP, _DOUT), jnp.float32),
            jax.ShapeDtypeStruct((_NP, _DOUT), jnp.float32),
        ],
    )(dp, h0)


def _final_body(h_ref, sd_ref, t_ref, *refs):
    g_refs, hid_ref = refs[:_K], refs[_K]
    s = t_ref[0, 1] * g_refs[0][...]
    for k in range(1, _K):
        s = s + t_ref[0, k + 1] * g_refs[k][...]
    hid_ref[...] = t_ref[0, 0] * h_ref[...] + sd_ref[...] * s


def _final(h0, sd, tvec, gs):
    return pl.pallas_call(
        _final_body,
        grid=(_NP // _EW_R,),
        in_specs=[
            pl.BlockSpec((_EW_R, _DOUT), lambda i: (i, 0)),
            pl.BlockSpec((_EW_R, _DOUT), lambda i: (i, 0)),
            pl.BlockSpec((1, _K + 1), lambda i: (0, 0)),
        ] + [pl.BlockSpec((_EW_R, _DOUT), lambda i: (i, 0)) for _ in range(_K)],
        out_specs=pl.BlockSpec((_EW_R, _DOUT), lambda i: (i, 0)),
        out_shape=jax.ShapeDtypeStruct((_NP, _DOUT), jnp.float32),
    )(h0, sd, tvec, *gs)


def kernel(x, edge_index, W1, b1, W2, b2, temp):
    row2 = edge_index[0].reshape(_NW, _EPW)
    col2 = edge_index[1].reshape(_NW, _EPW)

    xp = jnp.pad(x, ((0, _NP - _N), (0, 0)))
    h0 = _mlp(xp, W1, b1, W2, b2)
    dp = _hist_sc(col2.reshape(_NW, _HCH, _HC))
    g, d2, sd = _prep(dp, h0)

    ar, ac, br, bc = _part_sc(row2, col2)

    def slabs(a, b):
        s = jnp.stack([a, b]).reshape(_NC, _NS, _CH, _C)
        return jnp.concatenate([s, s[:, :, :2]], axis=2)

    rows = slabs(ar, br)
    cols = slabs(ac, bc)

    gs = []
    for _ in range(_K):
        g = _hop_sc(g, rows, cols, d2)
        gs.append(g)

    hidden = _final(h0, sd, temp.reshape(1, _K + 1), gs)
    return hidden[:_N]


# speedup vs baseline: 8.1009x; 1.0488x over previous
import functools

import jax
import jax.numpy as jnp
from jax import lax
from jax.experimental import pallas as pl
from jax.experimental.pallas import tpu as pltpu
from jax.experimental.pallas import tpu_sc as plsc

_N = 10000
_NP = 10240
_NH = 5120
_E = 320000
_DIN = 128
_DH = 128
_DOUT = 64
_K = 10

_NC = 2
_NS = 16
_NW = _NC * _NS
_EPW = _E // _NW

_W = 5520
_DUMC = 5500
_ACCR = 5760
_ZSTR = _ACCR // _NS
_CSTR = _NH // _NS
_SUB = 4
_CSUB = _CSTR // _SUB

_C = 120
_CH = 2 * _W // _C
_CHP = _CH + 2

_HD = 16
_HC = 125
_HCH = _EPW // _HC
_HRPT = _NP // _NS
_ZR = 128

_mesh = plsc.VectorSubcoreMesh(core_axis_name="c", subcore_axis_name="s")
_sc_params = pltpu.CompilerParams(use_tc_tiling_on_sc=False)
_sc_params_nlp = pltpu.CompilerParams(use_tc_tiling_on_sc=False,
                                      needs_layout_passes=False)


def _fill_vmem_2d(ref, rows, cols, vec):
    def body(i, carry):
        for c in range(cols // 16):
            ref[i, pl.ds(c * 16, 16)] = vec
        return carry

    lax.fori_loop(0, rows, body, 0)


@functools.partial(
    pl.kernel,
    mesh=_mesh,
    out_type=jax.ShapeDtypeStruct((_NC, _NP, _HD), jnp.float32),
    scratch_types=[
        pltpu.VMEM((_HCH, _HC), jnp.int32),
        pltpu.VMEM((_HC, _HD), jnp.float32),
        pltpu.VMEM((_ZR, _HD), jnp.float32),
        pltpu.VMEM_SHARED((_NP, _HD), jnp.float32),
    ],
    compiler_params=_sc_params,
)
def _hist_sc(col_hbm, out_hbm, colv, onesb, zbuf, acc):
    cid = lax.axis_index("c")
    sid = lax.axis_index("s")
    wid = cid * _NS + sid

    _fill_vmem_2d(zbuf, _ZR, _HD, jnp.zeros((16,), jnp.float32))
    _fill_vmem_2d(onesb, _HC, _HD, jnp.ones((16,), jnp.float32))

    for z in range(_HRPT // _ZR):
        pltpu.sync_copy(zbuf, acc.at[pl.ds(sid * _HRPT + z * _ZR, _ZR)])
    plsc.subcore_barrier()

    pltpu.sync_copy(col_hbm.at[wid], colv)

    def chunk(j, carry):
        pltpu.sync_copy(onesb, acc.at[colv.at[j]], add=True)
        return carry

    lax.fori_loop(0, _HCH, chunk, 0)

    plsc.subcore_barrier()
    pltpu.sync_copy(
        acc.at[pl.ds(sid * _HRPT, _HRPT)],
        out_hbm.at[cid, pl.ds(sid * _HRPT, _HRPT)],
    )


@functools.partial(
    pl.kernel,
    mesh=_mesh,
    out_type=[jax.ShapeDtypeStruct((_NW, _W), jnp.int32) for _ in range(4)],
    scratch_types=[
        pltpu.VMEM((_EPW,), jnp.int32),
        pltpu.VMEM((_EPW,), jnp.int32),
        pltpu.VMEM((_W,), jnp.int32),
        pltpu.VMEM((_W,), jnp.int32),
        pltpu.VMEM((_W,), jnp.int32),
        pltpu.VMEM((_W,), jnp.int32),
        pltpu.VMEM((32,), jnp.int32),
    ],
    compiler_params=_sc_params_nlp,
)
def _part_sc(row_hbm, col_hbm, ar_hbm, ac_hbm, br_hbm, bc_hbm,
             rowf, colf, lar, lac, lbr, lbc, offs):
    cid = lax.axis_index("c")
    sid = lax.axis_index("s")
    wid = cid * _NS + sid

    pltpu.sync_copy(row_hbm.at[wid], rowf)
    pltpu.sync_copy(col_hbm.at[wid], colf)

    lane = lax.iota(jnp.int32, 16)

    def prefill(i, carry):
        spread = i * 16 + lane
        lar[pl.ds(i * 16, 16)] = spread & 1023
        lac[pl.ds(i * 16, 16)] = _NH + (spread & 511)
        lbr[pl.ds(i * 16, 16)] = spread & 1023
        lbc[pl.ds(i * 16, 16)] = _NH + (spread & 511)
        return carry

    lax.fori_loop(0, _W // 16, prefill, 0)

    offs[pl.ds(0, 16)] = jnp.zeros((16,), jnp.int32)
    offs[pl.ds(16, 16)] = jnp.zeros((16,), jnp.int32)
    lane1 = lax.iota(jnp.int32, 16) + 1

    def step(k, carry):
        rv = rowf[pl.ds(k * 16, 16)]
        cv = colf[pl.ds(k * 16, 16)]
        ma = cv < _NH
        mb = jnp.logical_not(ma)
        mai = jnp.where(ma, 1, 0)
        incla = plsc.cumsum(mai)
        inclb = lane1 - incla
        offa = offs[pl.ds(0, 16)]
        offb = offs[pl.ds(16, 16)]
        plsc.store_scatter(lar, [offa + incla - mai], rv, mask=ma)
        plsc.store_scatter(lac, [offa + incla - mai], cv, mask=ma)
        exclb = inclb - jnp.where(mb, 1, 0)
        plsc.store_scatter(lbr, [offb + exclb], rv, mask=mb)
        plsc.store_scatter(lbc, [offb + exclb], cv - _NH, mask=mb)
        tota = plsc.cummax(lax.rev(incla, (0,)))
        offs[pl.ds(0, 16)] = jnp.minimum(offa + tota, _W - 16)
        offs[pl.ds(16, 16)] = jnp.minimum(offb + 16 - tota, _W - 16)
        return carry

    lax.fori_loop(0, _EPW // 16, step, 0)

    pltpu.sync_copy(lar, ar_hbm.at[wid])
    pltpu.sync_copy(lac, ac_hbm.at[wid])
    pltpu.sync_copy(lbr, br_hbm.at[wid])
    pltpu.sync_copy(lbc, bc_hbm.at[wid])


@functools.partial(
    pl.kernel,
    mesh=_mesh,
    out_type=jax.ShapeDtypeStruct((_NP, _DOUT), jnp.float32),
    scratch_types=[
        pltpu.VMEM((_CHP, _C), jnp.int32),
        pltpu.VMEM((_CHP, _C), jnp.int32),
        pltpu.VMEM((_C, _DOUT), jnp.float32),
        pltpu.VMEM((_C, _DOUT), jnp.float32),
        pltpu.VMEM((_C, _DOUT), jnp.float32),
        pltpu.VMEM((_C, _DOUT), jnp.float32),
        pltpu.VMEM((_CSUB, _DOUT), jnp.float32),
        pltpu.VMEM((_CSUB, _DOUT), jnp.float32),
        pltpu.VMEM_SHARED((_ACCR, _DOUT), jnp.float32),
        [pltpu.SemaphoreType.DMA] * 4,
        [pltpu.SemaphoreType.DMA] * 4,
        pltpu.SemaphoreType.DMA,
    ],
    compiler_params=_sc_params,
)
def _hop_sc(g_hbm, row_hbm, col_hbm, d2_hbm, gout_hbm, rowv, colv,
            b0, b1, b2, b3, ab, db, acc, gsem, ssem, zsem):
    cid = lax.axis_index("c")
    sid = lax.axis_index("s")
    bufs = (b0, b1, b2, b3)

    def g_start(j, slot):
        pltpu.async_copy(g_hbm.at[rowv.at[j]], bufs[slot], gsem[slot])

    def g_wait(slot):
        pltpu.make_async_copy(g_hbm.at[rowv.at[0]], bufs[slot], gsem[slot]).wait()

    def s_start(j, slot):
        pltpu.async_copy(bufs[slot], acc.at[colv.at[j]], ssem[slot], add=True)

    def s_wait(slot):
        pltpu.make_async_copy(bufs[slot], acc.at[colv.at[0]], ssem[slot]).wait()

    pltpu.async_copy(
        g_hbm.at[pl.ds(cid * _NH + sid * _CSTR, _CSTR)],
        acc.at[pl.ds(sid * _CSTR, _CSTR)],
        zsem,
    )
    pltpu.sync_copy(row_hbm.at[cid, sid], rowv)
    pltpu.sync_copy(col_hbm.at[cid, sid], colv)
    pltpu.make_async_copy(
        g_hbm.at[pl.ds(cid * _NH, _CSTR)],
        acc.at[pl.ds(sid * _CSTR, _CSTR)],
        zsem,
    ).wait()
    plsc.subcore_barrier()

    g_start(0, 0)
    g_start(1, 1)
    g_wait(0); s_start(0, 0); g_start(2, 2)
    g_wait(1); s_start(1, 1); g_start(3, 3)
    g_wait(2); s_start(2, 2); s_wait(0); g_start(4, 0)
    g_wait(3); s_start(3, 3); s_wait(1); g_start(5, 1)

    def group(gi, carry):
        base = 4 * gi
        for b in range(4):
            j = base + b
            g_wait(b)
            s_start(j, b)
            s_wait((b + 2) % 4)
            g_start(j + 2, (b + 2) % 4)
        return carry

    lax.fori_loop(1, _CH // 4, group, 0)
    g_wait(0)
    g_wait(1)
    s_wait(2)
    s_wait(3)

    plsc.subcore_barrier()

    for sub in range(_SUB):
        loc = sid * _CSTR + sub * _CSUB
        gbase = cid * _NH + sid * _CSTR + sub * _CSUB
        pltpu.sync_copy(acc.at[pl.ds(loc, _CSUB)], ab)
        pltpu.sync_copy(d2_hbm.at[pl.ds(gbase, _CSUB)], db)

        def crow(i, carry):
            for q in range(_DOUT // 16):
                sl = pl.ds(q * 16, 16)
                ab[i, sl] = db[i, sl] * ab[i, sl]
            return carry

        lax.fori_loop(0, _CSUB, crow, 0)
        pltpu.sync_copy(ab, gout_hbm.at[pl.ds(gbase, _CSUB)])


def _mlp_body(x_ref, w1_ref, b1_ref, w2_ref, b2_ref, o_ref):
    h = jnp.dot(x_ref[...], w1_ref[...], preferred_element_type=jnp.float32)
    h = jnp.maximum(h + b1_ref[...], 0.0)
    o_ref[...] = (
        jnp.dot(h, w2_ref[...], preferred_element_type=jnp.float32) + b2_ref[...]
    )


_MLP_R = 1024


def _mlp(x, w1, b1, w2, b2):
    return pl.pallas_call(
        _mlp_body,
        grid=(_NP // _MLP_R,),
        in_specs=[
            pl.BlockSpec((_MLP_R, _DIN), lambda i: (i, 0)),
            pl.BlockSpec((_DIN, _DH), lambda i: (0, 0)),
            pl.BlockSpec((1, _DH), lambda i: (0, 0)),
            pl.BlockSpec((_DH, _DOUT), lambda i: (0, 0)),
            pl.BlockSpec((1, _DOUT), lambda i: (0, 0)),
        ],
        out_specs=pl.BlockSpec((_MLP_R, _DOUT), lambda i: (i, 0)),
        out_shape=jax.ShapeDtypeStruct((_NP, _DOUT), jnp.float32),
    )(x, w1, b1.reshape(1, _DH), w2, b2.reshape(1, _DOUT))


def _prep_body(dp_ref, h_ref, g_ref, d2_ref, sd_ref):
    deg = dp_ref[0] + dp_ref[1] + 1.0
    r = deg.shape[0]
    dinv = lax.rsqrt(deg)[:, 0:1]
    dinvb = jnp.broadcast_to(dinv, (r, _DOUT))
    g_ref[...] = dinvb * h_ref[...]
    d2_ref[...] = dinvb * dinvb
    sd_ref[...] = jnp.broadcast_to(jnp.sqrt(deg)[:, 0:1], (r, _DOUT))


_EW_R = 1024


def _prep(dp, h0):
    return pl.pallas_call(
        _prep_body,
        grid=(_NP // _EW_R,),
        in_specs=[
            pl.BlockSpec((_NC, _EW_R, _HD), lambda i: (0, i, 0)),
            pl.BlockSpec((_EW_R, _DOUT), lambda i: (i, 0)),
        ],
        out_specs=[
            pl.BlockSpec((_EW_R, _DOUT), lambda i: (i, 0)),
            pl.BlockSpec((_EW_R, _DOUT), lambda i: (i, 0)),
            pl.BlockSpec((_EW_R, _DOUT), lambda i: (i, 0)),
        ],
        out_shape=[
            jax.ShapeDtypeStruct((_NP, _DOUT), jnp.float32),
            jax.ShapeDtypeStruct((_NP, _DOUT), jnp.float32),
            jax.ShapeDtypeStruct((_NP, _DOUT), jnp.float32),
        ],
    )(dp, h0)


def _final_body(h_ref, sd_ref, t_ref, *refs):
    g_refs, hid_ref = refs[:_K], refs[_K]
    s = t_ref[0, 1] * g_refs[0][...]
    for k in range(1, _K):
        s = s + t_ref[0, k + 1] * g_refs[k][...]
    hid_ref[...] = t_ref[0, 0] * h_ref[...] + sd_ref[...] * s


def _final(h0, sd, tvec, gs):
    return pl.pallas_call(
        _final_body,
        grid=(_NP // _EW_R,),
        in_specs=[
            pl.BlockSpec((_EW_R, _DOUT), lambda i: (i, 0)),
            pl.BlockSpec((_EW_R, _DOUT), lambda i: (i, 0)),
            pl.BlockSpec((1, _K + 1), lambda i: (0, 0)),
        ] + [pl.BlockSpec((_EW_R, _DOUT), lambda i: (i, 0)) for _ in range(_K)],
        out_specs=pl.BlockSpec((_EW_R, _DOUT), lambda i: (i, 0)),
        out_shape=jax.ShapeDtypeStruct((_NP, _DOUT), jnp.float32),
    )(h0, sd, tvec, *gs)


def kernel(x, edge_index, W1, b1, W2, b2, temp):
    row2 = edge_index[0].reshape(_NW, _EPW)
    col2 = edge_index[1].reshape(_NW, _EPW)

    xp = jnp.pad(x, ((0, _NP - _N), (0, 0)))
    h0 = _mlp(xp, W1, b1, W2, b2)
    dp = _hist_sc(col2.reshape(_NW, _HCH, _HC))
    g, d2, sd = _prep(dp, h0)

    ar, ac, br, bc = _part_sc(row2, col2)

    def slabs(a, b):
        s = jnp.stack([a, b]).reshape(_NC, _NS, _CH, _C)
        return jnp.concatenate([s, s[:, :, :2]], axis=2)

    rows = slabs(ar, br)
    cols = slabs(ac, bc)

    gs = []
    for _ in range(_K):
        g = _hop_sc(g, rows, cols, d2)
        gs.append(g)

    hidden = _final(h0, sd, temp.reshape(1, _K + 1), gs)
    return hidden[:_N]


# speedup vs baseline: 8.4280x; 1.0404x over previous
import functools

import jax
import jax.numpy as jnp
from jax import lax
from jax.experimental import pallas as pl
from jax.experimental.pallas import tpu as pltpu
from jax.experimental.pallas import tpu_sc as plsc

_N = 10000
_NP = 10240
_NH = 5120
_E = 320000
_DIN = 128
_DH = 128
_DOUT = 64
_K = 10

_NC = 2
_NS = 16
_NW = _NC * _NS
_EPW = _E // _NW

_W = 5520
_DUMC = 5500
_ACCR = 5760
_ZSTR = _ACCR // _NS
_CSTR = _NH // _NS
_SUB = 4
_CSUB = _CSTR // _SUB

_C = 120
_CH = 2 * _W // _C
_CHP = _CH + 2

_HD = 16
_HC = 125
_HCH = _EPW // _HC
_HRPT = _NP // _NS
_ZR = 128

_mesh = plsc.VectorSubcoreMesh(core_axis_name="c", subcore_axis_name="s")
_sc_params = pltpu.CompilerParams(use_tc_tiling_on_sc=False)
_sc_params_nlp = pltpu.CompilerParams(use_tc_tiling_on_sc=False,
                                      needs_layout_passes=False)


def _fill_vmem_2d(ref, rows, cols, vec):
    def body(i, carry):
        for c in range(cols // 16):
            ref[i, pl.ds(c * 16, 16)] = vec
        return carry

    lax.fori_loop(0, rows, body, 0)


@functools.partial(
    pl.kernel,
    mesh=_mesh,
    out_type=jax.ShapeDtypeStruct((_NC, _NP, _HD), jnp.float32),
    scratch_types=[
        pltpu.VMEM((_HCH, _HC), jnp.int32),
        pltpu.VMEM((_HC, _HD), jnp.float32),
        pltpu.VMEM((_ZR, _HD), jnp.float32),
        pltpu.VMEM_SHARED((_NP, _HD), jnp.float32),
    ],
    compiler_params=_sc_params,
)
def _hist_sc(col_hbm, out_hbm, colv, onesb, zbuf, acc):
    cid = lax.axis_index("c")
    sid = lax.axis_index("s")
    wid = cid * _NS + sid

    _fill_vmem_2d(zbuf, _ZR, _HD, jnp.zeros((16,), jnp.float32))
    _fill_vmem_2d(onesb, _HC, _HD, jnp.ones((16,), jnp.float32))

    for z in range(_HRPT // _ZR):
        pltpu.sync_copy(zbuf, acc.at[pl.ds(sid * _HRPT + z * _ZR, _ZR)])
    plsc.subcore_barrier()

    pltpu.sync_copy(col_hbm.at[wid], colv)

    def chunk(j, carry):
        pltpu.sync_copy(onesb, acc.at[colv.at[j]], add=True)
        return carry

    lax.fori_loop(0, _HCH, chunk, 0)

    plsc.subcore_barrier()
    pltpu.sync_copy(
        acc.at[pl.ds(sid * _HRPT, _HRPT)],
        out_hbm.at[cid, pl.ds(sid * _HRPT, _HRPT)],
    )


@functools.partial(
    pl.kernel,
    mesh=_mesh,
    out_type=[jax.ShapeDtypeStruct((_NW, _W), jnp.int32) for _ in range(4)],
    scratch_types=[
        pltpu.VMEM((_EPW,), jnp.int32),
        pltpu.VMEM((_EPW,), jnp.int32),
        pltpu.VMEM((_W,), jnp.int32),
        pltpu.VMEM((_W,), jnp.int32),
        pltpu.VMEM((_W,), jnp.int32),
        pltpu.VMEM((_W,), jnp.int32),
        pltpu.VMEM((32,), jnp.int32),
    ],
    compiler_params=_sc_params_nlp,
)
def _part_sc(row_hbm, col_hbm, ar_hbm, ac_hbm, br_hbm, bc_hbm,
             rowf, colf, lar, lac, lbr, lbc, offs):
    cid = lax.axis_index("c")
    sid = lax.axis_index("s")
    wid = cid * _NS + sid

    pltpu.sync_copy(row_hbm.at[wid], rowf)
    pltpu.sync_copy(col_hbm.at[wid], colf)

    lane = lax.iota(jnp.int32, 16)

    def prefill(i, carry):
        spread = i * 16 + lane
        lar[pl.ds(i * 16, 16)] = spread & 1023
        lac[pl.ds(i * 16, 16)] = _NH + (spread & 511)
        lbr[pl.ds(i * 16, 16)] = spread & 1023
        lbc[pl.ds(i * 16, 16)] = _NH + (spread & 511)
        return carry

    lax.fori_loop(0, _W // 16, prefill, 0)

    offs[pl.ds(0, 16)] = jnp.zeros((16,), jnp.int32)
    offs[pl.ds(16, 16)] = jnp.zeros((16,), jnp.int32)
    lane1 = lax.iota(jnp.int32, 16) + 1

    def step(k, carry):
        rv = rowf[pl.ds(k * 16, 16)]
        cv = colf[pl.ds(k * 16, 16)]
        ma = cv < _NH
        mb = jnp.logical_not(ma)
        mai = jnp.where(ma, 1, 0)
        incla = plsc.cumsum(mai)
        inclb = lane1 - incla
        offa = offs[pl.ds(0, 16)]
        offb = offs[pl.ds(16, 16)]
        plsc.store_scatter(lar, [offa + incla - mai], rv, mask=ma)
        plsc.store_scatter(lac, [offa + incla - mai], cv, mask=ma)
        exclb = inclb - jnp.where(mb, 1, 0)
        plsc.store_scatter(lbr, [offb + exclb], rv, mask=mb)
        plsc.store_scatter(lbc, [offb + exclb], cv - _NH, mask=mb)
        tota = plsc.cummax(lax.rev(incla, (0,)))
        offs[pl.ds(0, 16)] = jnp.minimum(offa + tota, _W - 16)
        offs[pl.ds(16, 16)] = jnp.minimum(offb + 16 - tota, _W - 16)
        return carry

    lax.fori_loop(0, _EPW // 16, step, 0)

    pltpu.sync_copy(lar, ar_hbm.at[wid])
    pltpu.sync_copy(lac, ac_hbm.at[wid])
    pltpu.sync_copy(lbr, br_hbm.at[wid])
    pltpu.sync_copy(lbc, bc_hbm.at[wid])


@functools.partial(
    pl.kernel,
    mesh=_mesh,
    out_type=jax.ShapeDtypeStruct((_NP, _DOUT), jnp.float32),
    scratch_types=[
        pltpu.VMEM((_CHP, _C), jnp.int32),
        pltpu.VMEM((_CHP, _C), jnp.int32),
        pltpu.VMEM((_C, _DOUT), jnp.float32),
        pltpu.VMEM((_C, _DOUT), jnp.float32),
        pltpu.VMEM((_C, _DOUT), jnp.float32),
        pltpu.VMEM((_C, _DOUT), jnp.float32),
        pltpu.VMEM((_CSTR, _DOUT), jnp.float32),
        pltpu.VMEM((_CSTR, _DOUT), jnp.float32),
        pltpu.VMEM_SHARED((_ACCR, _DOUT), jnp.float32),
        [pltpu.SemaphoreType.DMA] * 4,
        [pltpu.SemaphoreType.DMA] * 4,
        pltpu.SemaphoreType.DMA,
        pltpu.SemaphoreType.DMA,
    ],
    compiler_params=_sc_params,
)
def _hop_sc(g_hbm, row_hbm, col_hbm, d2_hbm, gout_hbm, rowv, colv,
            b0, b1, b2, b3, ab, db, acc, gsem, ssem, zsem, dsem):
    cid = lax.axis_index("c")
    sid = lax.axis_index("s")
    bufs = (b0, b1, b2, b3)

    def g_start(j, slot):
        pltpu.async_copy(g_hbm.at[rowv.at[j]], bufs[slot], gsem[slot])

    def g_wait(slot):
        pltpu.make_async_copy(g_hbm.at[rowv.at[0]], bufs[slot], gsem[slot]).wait()

    def s_start(j, slot):
        pltpu.async_copy(bufs[slot], acc.at[colv.at[j]], ssem[slot], add=True)

    def s_wait(slot):
        pltpu.make_async_copy(bufs[slot], acc.at[colv.at[0]], ssem[slot]).wait()

    pltpu.async_copy(
        g_hbm.at[pl.ds(cid * _NH + sid * _CSTR, _CSTR)],
        acc.at[pl.ds(sid * _CSTR, _CSTR)],
        zsem,
    )
    pltpu.async_copy(
        d2_hbm.at[pl.ds(cid * _NH + sid * _CSTR, _CSTR)], db, dsem
    )
    pltpu.sync_copy(row_hbm.at[cid, sid], rowv)
    pltpu.sync_copy(col_hbm.at[cid, sid], colv)
    pltpu.make_async_copy(
        g_hbm.at[pl.ds(cid * _NH, _CSTR)],
        acc.at[pl.ds(sid * _CSTR, _CSTR)],
        zsem,
    ).wait()
    plsc.subcore_barrier()

    g_start(0, 0)
    g_start(1, 1)
    g_wait(0); s_start(0, 0); g_start(2, 2)
    g_wait(1); s_start(1, 1); g_start(3, 3)
    g_wait(2); s_start(2, 2); s_wait(0); g_start(4, 0)
    g_wait(3); s_start(3, 3); s_wait(1); g_start(5, 1)

    def group(gi, carry):
        base = 4 * gi
        for b in range(4):
            j = base + b
            g_wait(b)
            s_start(j, b)
            s_wait((b + 2) % 4)
            g_start(j + 2, (b + 2) % 4)
        return carry

    lax.fori_loop(1, _CH // 4, group, 0)
    g_wait(0)
    g_wait(1)
    s_wait(2)
    s_wait(3)

    plsc.subcore_barrier()

    pltpu.make_async_copy(
        d2_hbm.at[pl.ds(cid * _NH, _CSTR)], db, dsem
    ).wait()
    pltpu.sync_copy(acc.at[pl.ds(sid * _CSTR, _CSTR)], ab)

    def crow(i, carry):
        for q in range(_DOUT // 16):
            sl = pl.ds(q * 16, 16)
            ab[i, sl] = db[i, sl] * ab[i, sl]
        return carry

    lax.fori_loop(0, _CSTR, crow, 0)
    pltpu.sync_copy(ab, gout_hbm.at[pl.ds(cid * _NH + sid * _CSTR, _CSTR)])


def _mlp_body(x_ref, w1_ref, b1_ref, w2_ref, b2_ref, o_ref):
    h = jnp.dot(x_ref[...], w1_ref[...], preferred_element_type=jnp.float32)
    h = jnp.maximum(h + b1_ref[...], 0.0)
    o_ref[...] = (
        jnp.dot(h, w2_ref[...], preferred_element_type=jnp.float32) + b2_ref[...]
    )


_MLP_R = 1024


def _mlp(x, w1, b1, w2, b2):
    return pl.pallas_call(
        _mlp_body,
        grid=(_NP // _MLP_R,),
        in_specs=[
            pl.BlockSpec((_MLP_R, _DIN), lambda i: (i, 0)),
            pl.BlockSpec((_DIN, _DH), lambda i: (0, 0)),
            pl.BlockSpec((1, _DH), lambda i: (0, 0)),
            pl.BlockSpec((_DH, _DOUT), lambda i: (0, 0)),
            pl.BlockSpec((1, _DOUT), lambda i: (0, 0)),
        ],
        out_specs=pl.BlockSpec((_MLP_R, _DOUT), lambda i: (i, 0)),
        out_shape=jax.ShapeDtypeStruct((_NP, _DOUT), jnp.float32),
    )(x, w1, b1.reshape(1, _DH), w2, b2.reshape(1, _DOUT))


def _prep_body(dp_ref, h_ref, g_ref, d2_ref, sd_ref):
    deg = dp_ref[0] + dp_ref[1] + 1.0
    r = deg.shape[0]
    dinv = lax.rsqrt(deg)[:, 0:1]
    dinvb = jnp.broadcast_to(dinv, (r, _DOUT))
    g_ref[...] = dinvb * h_ref[...]
    d2_ref[...] = dinvb * dinvb
    sd_ref[...] = jnp.broadcast_to(jnp.sqrt(deg)[:, 0:1], (r, _DOUT))


_EW_R = 1024


def _prep(dp, h0):
    return pl.pallas_call(
        _prep_body,
        grid=(_NP // _EW_R,),
        in_specs=[
            pl.BlockSpec((_NC, _EW_R, _HD), lambda i: (0, i, 0)),
            pl.BlockSpec((_EW_R, _DOUT), lambda i: (i, 0)),
        ],
        out_specs=[
            pl.BlockSpec((_EW_R, _DOUT), lambda i: (i, 0)),
            pl.BlockSpec((_EW_R, _DOUT), lambda i: (i, 0)),
            pl.BlockSpec((_EW_R, _DOUT), lambda i: (i, 0)),
        ],
        out_shape=[
            jax.ShapeDtypeStruct((_NP, _DOUT), jnp.float32),
            jax.ShapeDtypeStruct((_NP, _DOUT), jnp.float32),
            jax.ShapeDtypeStruct((_NP, _DOUT), jnp.float32),
        ],
    )(dp, h0)


def _final_body(h_ref, sd_ref, t_ref, *refs):
    g_refs, hid_ref = refs[:_K], refs[_K]
    s = t_ref[0, 1] * g_refs[0][...]
    for k in range(1, _K):
        s = s + t_ref[0, k + 1] * g_refs[k][...]
    hid_ref[...] = t_ref[0, 0] * h_ref[...] + sd_ref[...] * s


def _final(h0, sd, tvec, gs):
    return pl.pallas_call(
        _final_body,
        grid=(_NP // _EW_R,),
        in_specs=[
            pl.BlockSpec((_EW_R, _DOUT), lambda i: (i, 0)),
            pl.BlockSpec((_EW_R, _DOUT), lambda i: (i, 0)),
            pl.BlockSpec((1, _K + 1), lambda i: (0, 0)),
        ] + [pl.BlockSpec((_EW_R, _DOUT), lambda i: (i, 0)) for _ in range(_K)],
        out_specs=pl.BlockSpec((_EW_R, _DOUT), lambda i: (i, 0)),
        out_shape=jax.ShapeDtypeStruct((_NP, _DOUT), jnp.float32),
    )(h0, sd, tvec, *gs)


def kernel(x, edge_index, W1, b1, W2, b2, temp):
    row2 = edge_index[0].reshape(_NW, _EPW)
    col2 = edge_index[1].reshape(_NW, _EPW)

    xp = jnp.pad(x, ((0, _NP - _N), (0, 0)))
    h0 = _mlp(xp, W1, b1, W2, b2)
    dp = _hist_sc(col2.reshape(_NW, _HCH, _HC))
    g, d2, sd = _prep(dp, h0)

    ar, ac, br, bc = _part_sc(row2, col2)

    def slabs(a, b):
        s = jnp.stack([a, b]).reshape(_NC, _NS, _CH, _C)
        return jnp.concatenate([s, s[:, :, :2]], axis=2)

    rows = slabs(ar, br)
    cols = slabs(ac, bc)

    gs = []
    for _ in range(_K):
        g = _hop_sc(g, rows, cols, d2)
        gs.append(g)

    hidden = _final(h0, sd, temp.reshape(1, _K + 1), gs)
    return hidden[:_N]


# speedup vs baseline: 8.5377x; 1.0130x over previous
import functools

import jax
import jax.numpy as jnp
from jax import lax
from jax.experimental import pallas as pl
from jax.experimental.pallas import tpu as pltpu
from jax.experimental.pallas import tpu_sc as plsc

_N = 10000
_NP = 10240
_NH = 5120
_E = 320000
_DIN = 128
_DH = 128
_DOUT = 64
_K = 10

_NC = 2
_NS = 16
_NW = _NC * _NS
_EPW = _E // _NW

_W = 5520
_DUMC = 5500
_ACCR = 5760
_ZSTR = _ACCR // _NS
_CSTR = _NH // _NS
_SUB = 4
_CSUB = _CSTR // _SUB

_C = 120
_CH = 2 * _W // _C
_CHP = _CH + 2

_HD = 16
_HC = 125
_HCH = _EPW // _HC
_HRPT = _NP // _NS
_ZR = 128

_mesh = plsc.VectorSubcoreMesh(core_axis_name="c", subcore_axis_name="s")
_sc_params = pltpu.CompilerParams(use_tc_tiling_on_sc=False)
_sc_params_nlp = pltpu.CompilerParams(use_tc_tiling_on_sc=False,
                                      needs_layout_passes=False)


def _fill_vmem_2d(ref, rows, cols, vec):
    def body(i, carry):
        for c in range(cols // 16):
            ref[i, pl.ds(c * 16, 16)] = vec
        return carry

    lax.fori_loop(0, rows, body, 0)


@functools.partial(
    pl.kernel,
    mesh=_mesh,
    out_type=jax.ShapeDtypeStruct((_NC, _NP, _HD), jnp.float32),
    scratch_types=[
        pltpu.VMEM((_HCH, _HC), jnp.int32),
        pltpu.VMEM((_HC, _HD), jnp.float32),
        pltpu.VMEM((_ZR, _HD), jnp.float32),
        pltpu.VMEM_SHARED((_NP, _HD), jnp.float32),
    ],
    compiler_params=_sc_params,
)
def _hist_sc(col_hbm, out_hbm, colv, onesb, zbuf, acc):
    cid = lax.axis_index("c")
    sid = lax.axis_index("s")
    wid = cid * _NS + sid

    _fill_vmem_2d(zbuf, _ZR, _HD, jnp.zeros((16,), jnp.float32))
    _fill_vmem_2d(onesb, _HC, _HD, jnp.ones((16,), jnp.float32))

    for z in range(_HRPT // _ZR):
        pltpu.sync_copy(zbuf, acc.at[pl.ds(sid * _HRPT + z * _ZR, _ZR)])
    plsc.subcore_barrier()

    pltpu.sync_copy(col_hbm.at[wid], colv)

    def chunk(j, carry):
        pltpu.sync_copy(onesb, acc.at[colv.at[j]], add=True)
        return carry

    lax.fori_loop(0, _HCH, chunk, 0)

    plsc.subcore_barrier()
    pltpu.sync_copy(
        acc.at[pl.ds(sid * _HRPT, _HRPT)],
        out_hbm.at[cid, pl.ds(sid * _HRPT, _HRPT)],
    )


@functools.partial(
    pl.kernel,
    mesh=_mesh,
    out_type=[jax.ShapeDtypeStruct((_NW, _W), jnp.int32) for _ in range(4)]
    + [jax.ShapeDtypeStruct((_NC, _NP // 16, 16), jnp.float32)],
    scratch_types=[
        pltpu.VMEM((_EPW,), jnp.int32),
        pltpu.VMEM((_EPW,), jnp.int32),
        pltpu.VMEM((_W,), jnp.int32),
        pltpu.VMEM((_W,), jnp.int32),
        pltpu.VMEM((_W,), jnp.int32),
        pltpu.VMEM((_W,), jnp.int32),
        pltpu.VMEM((32,), jnp.int32),
        pltpu.VMEM((_NP // 16, 16), jnp.float32),
        pltpu.VMEM((5, 128), jnp.int32),
        pltpu.VMEM((40, 16), jnp.float32),
        pltpu.VMEM_SHARED((_NP // 16, 16), jnp.float32),
    ],
    compiler_params=_sc_params_nlp,
)
def _part_sc(row_hbm, col_hbm, ar_hbm, ac_hbm, br_hbm, bc_hbm, deg_hbm,
             rowf, colf, lar, lac, lbr, lbc, offs, degloc, idxv, zdeg, dacc):
    cid = lax.axis_index("c")
    sid = lax.axis_index("s")
    wid = cid * _NS + sid

    pltpu.sync_copy(row_hbm.at[wid], rowf)
    pltpu.sync_copy(col_hbm.at[wid], colf)

    lane = lax.iota(jnp.int32, 16)
    zvec = jnp.zeros((16,), jnp.float32)

    def zdl(i, carry):
        degloc[i, pl.ds(0, 16)] = zvec
        return carry

    lax.fori_loop(0, _NP // 16, zdl, 0)
    _fill_vmem_2d(zdeg, 40, 16, zvec)

    def fidx(i, carry):
        for q in range(8):
            idxv[i, pl.ds(q * 16, 16)] = i * 128 + q * 16 + lane
        return carry

    lax.fori_loop(0, 5, fidx, 0)
    pltpu.sync_copy(zdeg, dacc.at[pl.ds(sid * 40, 40)])
    plsc.subcore_barrier()

    def prefill(i, carry):
        spread = i * 16 + lane
        lar[pl.ds(i * 16, 16)] = spread & 1023
        lac[pl.ds(i * 16, 16)] = _NH + (spread & 511)
        lbr[pl.ds(i * 16, 16)] = spread & 1023
        lbc[pl.ds(i * 16, 16)] = _NH + (spread & 511)
        return carry

    lax.fori_loop(0, _W // 16, prefill, 0)

    offs[pl.ds(0, 16)] = jnp.zeros((16,), jnp.int32)
    offs[pl.ds(16, 16)] = jnp.zeros((16,), jnp.int32)
    lane1 = lax.iota(jnp.int32, 16) + 1

    def step(k, carry):
        rv = rowf[pl.ds(k * 16, 16)]
        cv = colf[pl.ds(k * 16, 16)]
        ma = cv < _NH
        mb = jnp.logical_not(ma)
        mai = jnp.where(ma, 1, 0)
        plsc.addupdate_scatter(
            degloc,
            [lax.shift_right_logical(cv, 4), cv & 15],
            jnp.ones((16,), jnp.float32),
        )
        incla = plsc.cumsum(mai)
        inclb = lane1 - incla
        offa = offs[pl.ds(0, 16)]
        offb = offs[pl.ds(16, 16)]
        plsc.store_scatter(lar, [offa + incla - mai], rv, mask=ma)
        plsc.store_scatter(lac, [offa + incla - mai], cv, mask=ma)
        exclb = inclb - jnp.where(mb, 1, 0)
        plsc.store_scatter(lbr, [offb + exclb], rv, mask=mb)
        plsc.store_scatter(lbc, [offb + exclb], cv - _NH, mask=mb)
        tota = plsc.cummax(lax.rev(incla, (0,)))
        offs[pl.ds(0, 16)] = jnp.minimum(offa + tota, _W - 16)
        offs[pl.ds(16, 16)] = jnp.minimum(offb + 16 - tota, _W - 16)
        return carry

    lax.fori_loop(0, _EPW // 16, step, 0)

    pltpu.sync_copy(lar, ar_hbm.at[wid])
    pltpu.sync_copy(lac, ac_hbm.at[wid])
    pltpu.sync_copy(lbr, br_hbm.at[wid])
    pltpu.sync_copy(lbc, bc_hbm.at[wid])

    for j in range(5):
        pltpu.sync_copy(
            degloc.at[pl.ds(j * 128, 128)], dacc.at[idxv.at[j]], add=True
        )
    plsc.subcore_barrier()
    pltpu.sync_copy(
        dacc.at[pl.ds(sid * 40, 40)], deg_hbm.at[cid, pl.ds(sid * 40, 40)]
    )


@functools.partial(
    pl.kernel,
    mesh=_mesh,
    out_type=jax.ShapeDtypeStruct((_NP, _DOUT), jnp.float32),
    scratch_types=[
        pltpu.VMEM((_CHP, _C), jnp.int32),
        pltpu.VMEM((_CHP, _C), jnp.int32),
        pltpu.VMEM((_C, _DOUT), jnp.float32),
        pltpu.VMEM((_C, _DOUT), jnp.float32),
        pltpu.VMEM((_C, _DOUT), jnp.float32),
        pltpu.VMEM((_C, _DOUT), jnp.float32),
        pltpu.VMEM((_CSTR, _DOUT), jnp.float32),
        pltpu.VMEM((_CSTR, _DOUT), jnp.float32),
        pltpu.VMEM_SHARED((_ACCR, _DOUT), jnp.float32),
        [pltpu.SemaphoreType.DMA] * 4,
        [pltpu.SemaphoreType.DMA] * 4,
        pltpu.SemaphoreType.DMA,
        pltpu.SemaphoreType.DMA,
    ],
    compiler_params=_sc_params,
)
def _hop_sc(g_hbm, row_hbm, col_hbm, d2_hbm, gout_hbm, rowv, colv,
            b0, b1, b2, b3, ab, db, acc, gsem, ssem, zsem, dsem):
    cid = lax.axis_index("c")
    sid = lax.axis_index("s")
    bufs = (b0, b1, b2, b3)

    def g_start(j, slot):
        pltpu.async_copy(g_hbm.at[rowv.at[j]], bufs[slot], gsem[slot])

    def g_wait(slot):
        pltpu.make_async_copy(g_hbm.at[rowv.at[0]], bufs[slot], gsem[slot]).wait()

    def s_start(j, slot):
        pltpu.async_copy(bufs[slot], acc.at[colv.at[j]], ssem[slot], add=True)

    def s_wait(slot):
        pltpu.make_async_copy(bufs[slot], acc.at[colv.at[0]], ssem[slot]).wait()

    pltpu.async_copy(
        g_hbm.at[pl.ds(cid * _NH + sid * _CSTR, _CSTR)],
        acc.at[pl.ds(sid * _CSTR, _CSTR)],
        zsem,
    )
    pltpu.async_copy(
        d2_hbm.at[pl.ds(cid * _NH + sid * _CSTR, _CSTR)], db, dsem
    )
    pltpu.sync_copy(row_hbm.at[cid, sid], rowv)
    pltpu.sync_copy(col_hbm.at[cid, sid], colv)
    pltpu.make_async_copy(
        g_hbm.at[pl.ds(cid * _NH, _CSTR)],
        acc.at[pl.ds(sid * _CSTR, _CSTR)],
        zsem,
    ).wait()
    plsc.subcore_barrier()

    g_start(0, 0)
    g_start(1, 1)
    g_wait(0); s_start(0, 0); g_start(2, 2)
    g_wait(1); s_start(1, 1); g_start(3, 3)
    g_wait(2); s_start(2, 2); s_wait(0); g_start(4, 0)
    g_wait(3); s_start(3, 3); s_wait(1); g_start(5, 1)

    def group(gi, carry):
        base = 4 * gi
        for b in range(4):
            j = base + b
            g_wait(b)
            s_start(j, b)
            s_wait((b + 2) % 4)
            g_start(j + 2, (b + 2) % 4)
        return carry

    lax.fori_loop(1, _CH // 4, group, 0)
    g_wait(0)
    g_wait(1)
    s_wait(2)
    s_wait(3)

    plsc.subcore_barrier()

    pltpu.make_async_copy(
        d2_hbm.at[pl.ds(cid * _NH, _CSTR)], db, dsem
    ).wait()
    pltpu.sync_copy(acc.at[pl.ds(sid * _CSTR, _CSTR)], ab)

    def crow(i, carry):
        for q in range(_DOUT // 16):
            sl = pl.ds(q * 16, 16)
            ab[i, sl] = db[i, sl] * ab[i, sl]
        return carry

    lax.fori_loop(0, _CSTR, crow, 0)
    pltpu.sync_copy(ab, gout_hbm.at[pl.ds(cid * _NH + sid * _CSTR, _CSTR)])


def _mlp_body(x_ref, w1_ref, b1_ref, w2_ref, b2_ref, o_ref):
    h = jnp.dot(x_ref[...], w1_ref[...], preferred_element_type=jnp.float32)
    h = jnp.maximum(h + b1_ref[...], 0.0)
    o_ref[...] = (
        jnp.dot(h, w2_ref[...], preferred_element_type=jnp.float32) + b2_ref[...]
    )


_MLP_R = 1024


def _mlp(x, w1, b1, w2, b2):
    return pl.pallas_call(
        _mlp_body,
        grid=(_NP // _MLP_R,),
        in_specs=[
            pl.BlockSpec((_MLP_R, _DIN), lambda i: (i, 0)),
            pl.BlockSpec((_DIN, _DH), lambda i: (0, 0)),
            pl.BlockSpec((1, _DH), lambda i: (0, 0)),
            pl.BlockSpec((_DH, _DOUT), lambda i: (0, 0)),
            pl.BlockSpec((1, _DOUT), lambda i: (0, 0)),
        ],
        out_specs=pl.BlockSpec((_MLP_R, _DOUT), lambda i: (i, 0)),
        out_shape=jax.ShapeDtypeStruct((_NP, _DOUT), jnp.float32),
    )(x, w1, b1.reshape(1, _DH), w2, b2.reshape(1, _DOUT))


def _prep_body(dp_ref, h_ref, g_ref, d2_ref, sd_ref):
    deg = (dp_ref[0] + dp_ref[1] + 1.0)[:, None]
    r = deg.shape[0]
    dinv = lax.rsqrt(deg)
    dinvb = jnp.broadcast_to(dinv, (r, _DOUT))
    g_ref[...] = dinvb * h_ref[...]
    d2_ref[...] = dinvb * dinvb
    sd_ref[...] = jnp.broadcast_to(jnp.sqrt(deg), (r, _DOUT))


_EW_R = 1024


def _prep(dp, h0):
    return pl.pallas_call(
        _prep_body,
        grid=(_NP // _EW_R,),
        in_specs=[
            pl.BlockSpec((_NC, _EW_R), lambda i: (0, i)),
            pl.BlockSpec((_EW_R, _DOUT), lambda i: (i, 0)),
        ],
        out_specs=[
            pl.BlockSpec((_EW_R, _DOUT), lambda i: (i, 0)),
            pl.BlockSpec((_EW_R, _DOUT), lambda i: (i, 0)),
            pl.BlockSpec((_EW_R, _DOUT), lambda i: (i, 0)),
        ],
        out_shape=[
            jax.ShapeDtypeStruct((_NP, _DOUT), jnp.float32),
            jax.ShapeDtypeStruct((_NP, _DOUT), jnp.float32),
            jax.ShapeDtypeStruct((_NP, _DOUT), jnp.float32),
        ],
    )(dp, h0)


def _final_body(h_ref, sd_ref, t_ref, *refs):
    g_refs, hid_ref = refs[:_K], refs[_K]
    s = t_ref[0, 1] * g_refs[0][...]
    for k in range(1, _K):
        s = s + t_ref[0, k + 1] * g_refs[k][...]
    hid_ref[...] = t_ref[0, 0] * h_ref[...] + sd_ref[...] * s


def _final(h0, sd, tvec, gs):
    return pl.pallas_call(
        _final_body,
        grid=(_NP // _EW_R,),
        in_specs=[
            pl.BlockSpec((_EW_R, _DOUT), lambda i: (i, 0)),
            pl.BlockSpec((_EW_R, _DOUT), lambda i: (i, 0)),
            pl.BlockSpec((1, _K + 1), lambda i: (0, 0)),
        ] + [pl.BlockSpec((_EW_R, _DOUT), lambda i: (i, 0)) for _ in range(_K)],
        out_specs=pl.BlockSpec((_EW_R, _DOUT), lambda i: (i, 0)),
        out_shape=jax.ShapeDtypeStruct((_NP, _DOUT), jnp.float32),
    )(h0, sd, tvec, *gs)


def kernel(x, edge_index, W1, b1, W2, b2, temp):
    row2 = edge_index[0].reshape(_NW, _EPW)
    col2 = edge_index[1].reshape(_NW, _EPW)

    xp = jnp.pad(x, ((0, _NP - _N), (0, 0)))
    h0 = _mlp(xp, W1, b1, W2, b2)
    ar, ac, br, bc, deg = _part_sc(row2, col2)
    g, d2, sd = _prep(deg.reshape(_NC, _NP), h0)

    def slabs(a, b):
        s = jnp.stack([a, b]).reshape(_NC, _NS, _CH, _C)
        return jnp.concatenate([s, s[:, :, :2]], axis=2)

    rows = slabs(ar, br)
    cols = slabs(ac, bc)

    gs = []
    for _ in range(_K):
        g = _hop_sc(g, rows, cols, d2)
        gs.append(g)

    hidden = _final(h0, sd, temp.reshape(1, _K + 1), gs)
    return hidden[:_N]


# speedup vs baseline: 8.5664x; 1.0034x over previous
import functools

import jax
import jax.numpy as jnp
from jax import lax
from jax.experimental import pallas as pl
from jax.experimental.pallas import tpu as pltpu
from jax.experimental.pallas import tpu_sc as plsc

_N = 10000
_NP = 10240
_NH = 5120
_E = 320000
_DIN = 128
_DH = 128
_DOUT = 64
_K = 10

_NC = 2
_NS = 16
_NW = _NC * _NS
_EPW = _E // _NW

_W = 5520
_DUMC = 5500
_ACCR = 5760
_ZSTR = _ACCR // _NS
_CSTR = _NH // _NS
_SUB = 4
_CSUB = _CSTR // _SUB

_C = 120
_CH = 2 * _W // _C
_CHP = _CH + 2

_HD = 16
_HC = 125
_HCH = _EPW // _HC
_HRPT = _NP // _NS
_ZR = 128

_mesh = plsc.VectorSubcoreMesh(core_axis_name="c", subcore_axis_name="s")
_sc_params = pltpu.CompilerParams(use_tc_tiling_on_sc=False)
_sc_params_nlp = pltpu.CompilerParams(use_tc_tiling_on_sc=False,
                                      needs_layout_passes=False)


def _fill_vmem_2d(ref, rows, cols, vec):
    def body(i, carry):
        for c in range(cols // 16):
            ref[i, pl.ds(c * 16, 16)] = vec
        return carry

    lax.fori_loop(0, rows, body, 0)


@functools.partial(
    pl.kernel,
    mesh=_mesh,
    out_type=jax.ShapeDtypeStruct((_NC, _NP, _HD), jnp.float32),
    scratch_types=[
        pltpu.VMEM((_HCH, _HC), jnp.int32),
        pltpu.VMEM((_HC, _HD), jnp.float32),
        pltpu.VMEM((_ZR, _HD), jnp.float32),
        pltpu.VMEM_SHARED((_NP, _HD), jnp.float32),
    ],
    compiler_params=_sc_params,
)
def _hist_sc(col_hbm, out_hbm, colv, onesb, zbuf, acc):
    cid = lax.axis_index("c")
    sid = lax.axis_index("s")
    wid = cid * _NS + sid

    _fill_vmem_2d(zbuf, _ZR, _HD, jnp.zeros((16,), jnp.float32))
    _fill_vmem_2d(onesb, _HC, _HD, jnp.ones((16,), jnp.float32))

    for z in range(_HRPT // _ZR):
        pltpu.sync_copy(zbuf, acc.at[pl.ds(sid * _HRPT + z * _ZR, _ZR)])
    plsc.subcore_barrier()

    pltpu.sync_copy(col_hbm.at[wid], colv)

    def chunk(j, carry):
        pltpu.sync_copy(onesb, acc.at[colv.at[j]], add=True)
        return carry

    lax.fori_loop(0, _HCH, chunk, 0)

    plsc.subcore_barrier()
    pltpu.sync_copy(
        acc.at[pl.ds(sid * _HRPT, _HRPT)],
        out_hbm.at[cid, pl.ds(sid * _HRPT, _HRPT)],
    )


@functools.partial(
    pl.kernel,
    mesh=_mesh,
    out_type=[jax.ShapeDtypeStruct((_NW, _W), jnp.int32) for _ in range(4)]
    + [jax.ShapeDtypeStruct((_NC, _NP // 16, 16), jnp.float32)],
    scratch_types=[
        pltpu.VMEM((_EPW,), jnp.int32),
        pltpu.VMEM((_EPW,), jnp.int32),
        pltpu.VMEM((_W,), jnp.int32),
        pltpu.VMEM((_W,), jnp.int32),
        pltpu.VMEM((_W,), jnp.int32),
        pltpu.VMEM((_W,), jnp.int32),
        pltpu.VMEM((32,), jnp.int32),
        pltpu.VMEM((_NP // 16, 16), jnp.float32),
        pltpu.VMEM((5, 128), jnp.int32),
        pltpu.VMEM((40, 16), jnp.float32),
        pltpu.VMEM_SHARED((_NP // 16, 16), jnp.float32),
    ],
    compiler_params=_sc_params_nlp,
)
def _part_sc(row_hbm, col_hbm, ar_hbm, ac_hbm, br_hbm, bc_hbm, deg_hbm,
             rowf, colf, lar, lac, lbr, lbc, offs, degloc, idxv, zdeg, dacc):
    cid = lax.axis_index("c")
    sid = lax.axis_index("s")
    wid = cid * _NS + sid

    pltpu.sync_copy(row_hbm.at[wid], rowf)
    pltpu.sync_copy(col_hbm.at[wid], colf)

    lane = lax.iota(jnp.int32, 16)
    zvec = jnp.zeros((16,), jnp.float32)

    def zdl(i, carry):
        degloc[i, pl.ds(0, 16)] = zvec
        return carry

    lax.fori_loop(0, _NP // 16, zdl, 0)
    _fill_vmem_2d(zdeg, 40, 16, zvec)

    def fidx(i, carry):
        for q in range(8):
            idxv[i, pl.ds(q * 16, 16)] = i * 128 + q * 16 + lane
        return carry

    lax.fori_loop(0, 5, fidx, 0)
    pltpu.sync_copy(zdeg, dacc.at[pl.ds(sid * 40, 40)])
    plsc.subcore_barrier()

    def prefill(i, carry):
        spread = i * 16 + lane
        lar[pl.ds(i * 16, 16)] = spread & 1023
        lac[pl.ds(i * 16, 16)] = _NH + (spread & 511)
        lbr[pl.ds(i * 16, 16)] = spread & 1023
        lbc[pl.ds(i * 16, 16)] = _NH + (spread & 511)
        return carry

    lax.fori_loop(0, _W // 16, prefill, 0)

    offs[pl.ds(0, 16)] = jnp.zeros((16,), jnp.int32)
    offs[pl.ds(16, 16)] = jnp.zeros((16,), jnp.int32)
    lane1 = lax.iota(jnp.int32, 16) + 1

    ones16 = jnp.ones((16,), jnp.float32)

    def halfstep(base, offa, offb):
        rv = rowf[pl.ds(base, 16)]
        cv = colf[pl.ds(base, 16)]
        ma = cv < _NH
        mb = jnp.logical_not(ma)
        mai = jnp.where(ma, 1, 0)
        plsc.addupdate_scatter(
            degloc, [lax.shift_right_logical(cv, 4), cv & 15], ones16
        )
        incla = plsc.cumsum(mai)
        plsc.store_scatter(lar, [offa + incla - mai], rv, mask=ma)
        plsc.store_scatter(lac, [offa + incla - mai], cv, mask=ma)
        exclb = (lane1 - incla) - jnp.where(mb, 1, 0)
        plsc.store_scatter(lbr, [offb + exclb], rv, mask=mb)
        plsc.store_scatter(lbc, [offb + exclb], cv - _NH, mask=mb)
        tota = plsc.cummax(lax.rev(incla, (0,)))
        return tota

    def step(k, carry):
        offa = offs[pl.ds(0, 16)]
        offb = offs[pl.ds(16, 16)]
        tota1 = halfstep(k * 32, offa, offb)
        tota2 = halfstep(k * 32 + 16, offa + tota1, offb + 16 - tota1)
        offs[pl.ds(0, 16)] = jnp.minimum(offa + tota1 + tota2, _W - 32)
        offs[pl.ds(16, 16)] = jnp.minimum(offb + 32 - tota1 - tota2, _W - 32)
        return carry

    lax.fori_loop(0, _EPW // 32, step, 0)
    offa = offs[pl.ds(0, 16)]
    offb = offs[pl.ds(16, 16)]
    halfstep(_EPW - 16, offa, offb)

    pltpu.sync_copy(lar, ar_hbm.at[wid])
    pltpu.sync_copy(lac, ac_hbm.at[wid])
    pltpu.sync_copy(lbr, br_hbm.at[wid])
    pltpu.sync_copy(lbc, bc_hbm.at[wid])

    for j in range(5):
        pltpu.sync_copy(
            degloc.at[pl.ds(j * 128, 128)], dacc.at[idxv.at[j]], add=True
        )
    plsc.subcore_barrier()
    pltpu.sync_copy(
        dacc.at[pl.ds(sid * 40, 40)], deg_hbm.at[cid, pl.ds(sid * 40, 40)]
    )


@functools.partial(
    pl.kernel,
    mesh=_mesh,
    out_type=jax.ShapeDtypeStruct((_NP, _DOUT), jnp.float32),
    scratch_types=[
        pltpu.VMEM((_CHP, _C), jnp.int32),
        pltpu.VMEM((_CHP, _C), jnp.int32),
        pltpu.VMEM((_C, _DOUT), jnp.float32),
        pltpu.VMEM((_C, _DOUT), jnp.float32),
        pltpu.VMEM((_C, _DOUT), jnp.float32),
        pltpu.VMEM((_C, _DOUT), jnp.float32),
        pltpu.VMEM((_CSTR, _DOUT), jnp.float32),
        pltpu.VMEM((_CSTR, _DOUT), jnp.float32),
        pltpu.VMEM_SHARED((_ACCR, _DOUT), jnp.float32),
        [pltpu.SemaphoreType.DMA] * 4,
        [pltpu.SemaphoreType.DMA] * 4,
        pltpu.SemaphoreType.DMA,
        pltpu.SemaphoreType.DMA,
    ],
    compiler_params=_sc_params,
)
def _hop_sc(g_hbm, row_hbm, col_hbm, d2_hbm, gout_hbm, rowv, colv,
            b0, b1, b2, b3, ab, db, acc, gsem, ssem, zsem, dsem):
    cid = lax.axis_index("c")
    sid = lax.axis_index("s")
    bufs = (b0, b1, b2, b3)

    def g_start(j, slot):
        pltpu.async_copy(g_hbm.at[rowv.at[j]], bufs[slot], gsem[slot])

    def g_wait(slot):
        pltpu.make_async_copy(g_hbm.at[rowv.at[0]], bufs[slot], gsem[slot]).wait()

    def s_start(j, slot):
        pltpu.async_copy(bufs[slot], acc.at[colv.at[j]], ssem[slot], add=True)

    def s_wait(slot):
        pltpu.make_async_copy(bufs[slot], acc.at[colv.at[0]], ssem[slot]).wait()

    pltpu.async_copy(
        g_hbm.at[pl.ds(cid * _NH + sid * _CSTR, _CSTR)],
        acc.at[pl.ds(sid * _CSTR, _CSTR)],
        zsem,
    )
    pltpu.async_copy(
        d2_hbm.at[pl.ds(cid * _NH + sid * _CSTR, _CSTR)], db, dsem
    )
    pltpu.sync_copy(row_hbm.at[cid, sid], rowv)
    pltpu.sync_copy(col_hbm.at[cid, sid], colv)
    pltpu.make_async_copy(
        g_hbm.at[pl.ds(cid * _NH, _CSTR)],
        acc.at[pl.ds(sid * _CSTR, _CSTR)],
        zsem,
    ).wait()
    plsc.subcore_barrier()

    g_start(0, 0)
    g_start(1, 1)
    g_wait(0); s_start(0, 0); g_start(2, 2)
    g_wait(1); s_start(1, 1); g_start(3, 3)
    g_wait(2); s_start(2, 2); s_wait(0); g_start(4, 0)
    g_wait(3); s_start(3, 3); s_wait(1); g_start(5, 1)

    def group(gi, carry):
        base = 4 * gi
        for b in range(4):
            j = base + b
            g_wait(b)
            s_start(j, b)
            s_wait((b + 2) % 4)
            g_start(j + 2, (b + 2) % 4)
        return carry

    lax.fori_loop(1, _CH // 4, group, 0)
    g_wait(0)
    g_wait(1)
    s_wait(2)
    s_wait(3)

    plsc.subcore_barrier()

    pltpu.make_async_copy(
        d2_hbm.at[pl.ds(cid * _NH, _CSTR)], db, dsem
    ).wait()
    pltpu.sync_copy(acc.at[pl.ds(sid * _CSTR, _CSTR)], ab)

    def crow(i, carry):
        for q in range(_DOUT // 16):
            sl = pl.ds(q * 16, 16)
            ab[i, sl] = db[i, sl] * ab[i, sl]
        return carry

    lax.fori_loop(0, _CSTR, crow, 0)
    pltpu.sync_copy(ab, gout_hbm.at[pl.ds(cid * _NH + sid * _CSTR, _CSTR)])


def _mlp_body(x_ref, w1_ref, b1_ref, w2_ref, b2_ref, o_ref):
    h = jnp.dot(x_ref[...], w1_ref[...], preferred_element_type=jnp.float32)
    h = jnp.maximum(h + b1_ref[...], 0.0)
    o_ref[...] = (
        jnp.dot(h, w2_ref[...], preferred_element_type=jnp.float32) + b2_ref[...]
    )


_MLP_R = 1024


def _mlp(x, w1, b1, w2, b2):
    return pl.pallas_call(
        _mlp_body,
        grid=(_NP // _MLP_R,),
        in_specs=[
            pl.BlockSpec((_MLP_R, _DIN), lambda i: (i, 0)),
            pl.BlockSpec((_DIN, _DH), lambda i: (0, 0)),
            pl.BlockSpec((1, _DH), lambda i: (0, 0)),
            pl.BlockSpec((_DH, _DOUT), lambda i: (0, 0)),
            pl.BlockSpec((1, _DOUT), lambda i: (0, 0)),
        ],
        out_specs=pl.BlockSpec((_MLP_R, _DOUT), lambda i: (i, 0)),
        out_shape=jax.ShapeDtypeStruct((_NP, _DOUT), jnp.float32),
    )(x, w1, b1.reshape(1, _DH), w2, b2.reshape(1, _DOUT))


def _prep_body(dp_ref, h_ref, g_ref, d2_ref, sd_ref):
    deg = (dp_ref[0] + dp_ref[1] + 1.0)[:, None]
    r = deg.shape[0]
    dinv = lax.rsqrt(deg)
    dinvb = jnp.broadcast_to(dinv, (r, _DOUT))
    g_ref[...] = dinvb * h_ref[...]
    d2_ref[...] = dinvb * dinvb
    sd_ref[...] = jnp.broadcast_to(jnp.sqrt(deg), (r, _DOUT))


_EW_R = 1024


def _prep(dp, h0):
    return pl.pallas_call(
        _prep_body,
        grid=(_NP // _EW_R,),
        in_specs=[
            pl.BlockSpec((_NC, _EW_R), lambda i: (0, i)),
            pl.BlockSpec((_EW_R, _DOUT), lambda i: (i, 0)),
        ],
        out_specs=[
            pl.BlockSpec((_EW_R, _DOUT), lambda i: (i, 0)),
            pl.BlockSpec((_EW_R, _DOUT), lambda i: (i, 0)),
            pl.BlockSpec((_EW_R, _DOUT), lambda i: (i, 0)),
        ],
        out_shape=[
            jax.ShapeDtypeStruct((_NP, _DOUT), jnp.float32),
            jax.ShapeDtypeStruct((_NP, _DOUT), jnp.float32),
            jax.ShapeDtypeStruct((_NP, _DOUT), jnp.float32),
        ],
    )(dp, h0)


def _final_body(h_ref, sd_ref, t_ref, *refs):
    g_refs, hid_ref = refs[:_K], refs[_K]
    s = t_ref[0, 1] * g_refs[0][...]
    for k in range(1, _K):
        s = s + t_ref[0, k + 1] * g_refs[k][...]
    hid_ref[...] = t_ref[0, 0] * h_ref[...] + sd_ref[...] * s


def _final(h0, sd, tvec, gs):
    return pl.pallas_call(
        _final_body,
        grid=(_NP // _EW_R,),
        in_specs=[
            pl.BlockSpec((_EW_R, _DOUT), lambda i: (i, 0)),
            pl.BlockSpec((_EW_R, _DOUT), lambda i: (i, 0)),
            pl.BlockSpec((1, _K + 1), lambda i: (0, 0)),
        ] + [pl.BlockSpec((_EW_R, _DOUT), lambda i: (i, 0)) for _ in range(_K)],
        out_specs=pl.BlockSpec((_EW_R, _DOUT), lambda i: (i, 0)),
        out_shape=jax.ShapeDtypeStruct((_NP, _DOUT), jnp.float32),
    )(h0, sd, tvec, *gs)


def kernel(x, edge_index, W1, b1, W2, b2, temp):
    row2 = edge_index[0].reshape(_NW, _EPW)
    col2 = edge_index[1].reshape(_NW, _EPW)

    xp = jnp.pad(x, ((0, _NP - _N), (0, 0)))
    h0 = _mlp(xp, W1, b1, W2, b2)
    ar, ac, br, bc, deg = _part_sc(row2, col2)
    g, d2, sd = _prep(deg.reshape(_NC, _NP), h0)

    def slabs(a, b):
        s = jnp.stack([a, b]).reshape(_NC, _NS, _CH, _C)
        return jnp.concatenate([s, s[:, :, :2]], axis=2)

    rows = slabs(ar, br)
    cols = slabs(ac, bc)

    gs = []
    for _ in range(_K):
        g = _hop_sc(g, rows, cols, d2)
        gs.append(g)

    hidden = _final(h0, sd, temp.reshape(1, _K + 1), gs)
    return hidden[:_N]


# speedup vs baseline: 8.6130x; 1.0054x over previous
import functools

import jax
import jax.numpy as jnp
from jax import lax
from jax.experimental import pallas as pl
from jax.experimental.pallas import tpu as pltpu
from jax.experimental.pallas import tpu_sc as plsc

_N = 10000
_NP = 10240
_NH = 5120
_E = 320000
_DIN = 128
_DH = 128
_DOUT = 64
_K = 10

_NC = 2
_NS = 16
_NW = _NC * _NS
_EPW = _E // _NW

_W = 5520
_DUMC = 5500
_ACCR = 5760
_ZSTR = _ACCR // _NS
_CSTR = _NH // _NS
_SUB = 4
_CSUB = _CSTR // _SUB

_C = 120
_CH = 2 * _W // _C
_CHP = _CH + 2

_HD = 16
_HC = 125
_HCH = _EPW // _HC
_HRPT = _NP // _NS
_ZR = 128

_mesh = plsc.VectorSubcoreMesh(core_axis_name="c", subcore_axis_name="s")
_sc_params = pltpu.CompilerParams(use_tc_tiling_on_sc=False)
_sc_params_nlp = pltpu.CompilerParams(use_tc_tiling_on_sc=False,
                                      needs_layout_passes=False)


def _fill_vmem_2d(ref, rows, cols, vec):
    def body(i, carry):
        for c in range(cols // 16):
            ref[i, pl.ds(c * 16, 16)] = vec
        return carry

    lax.fori_loop(0, rows, body, 0)


@functools.partial(
    pl.kernel,
    mesh=_mesh,
    out_type=jax.ShapeDtypeStruct((_NC, _NP, _HD), jnp.float32),
    scratch_types=[
        pltpu.VMEM((_HCH, _HC), jnp.int32),
        pltpu.VMEM((_HC, _HD), jnp.float32),
        pltpu.VMEM((_ZR, _HD), jnp.float32),
        pltpu.VMEM_SHARED((_NP, _HD), jnp.float32),
    ],
    compiler_params=_sc_params,
)
def _hist_sc(col_hbm, out_hbm, colv, onesb, zbuf, acc):
    cid = lax.axis_index("c")
    sid = lax.axis_index("s")
    wid = cid * _NS + sid

    _fill_vmem_2d(zbuf, _ZR, _HD, jnp.zeros((16,), jnp.float32))
    _fill_vmem_2d(onesb, _HC, _HD, jnp.ones((16,), jnp.float32))

    for z in range(_HRPT // _ZR):
        pltpu.sync_copy(zbuf, acc.at[pl.ds(sid * _HRPT + z * _ZR, _ZR)])
    plsc.subcore_barrier()

    pltpu.sync_copy(col_hbm.at[wid], colv)

    def chunk(j, carry):
        pltpu.sync_copy(onesb, acc.at[colv.at[j]], add=True)
        return carry

    lax.fori_loop(0, _HCH, chunk, 0)

    plsc.subcore_barrier()
    pltpu.sync_copy(
        acc.at[pl.ds(sid * _HRPT, _HRPT)],
        out_hbm.at[cid, pl.ds(sid * _HRPT, _HRPT)],
    )


@functools.partial(
    pl.kernel,
    mesh=_mesh,
    out_type=[jax.ShapeDtypeStruct((_NW, _W), jnp.int32) for _ in range(4)]
    + [jax.ShapeDtypeStruct((_NC, _NP // 16, 16), jnp.float32)],
    scratch_types=[
        pltpu.VMEM((_EPW,), jnp.int32),
        pltpu.VMEM((_EPW,), jnp.int32),
        pltpu.VMEM((_W,), jnp.int32),
        pltpu.VMEM((_W,), jnp.int32),
        pltpu.VMEM((_W,), jnp.int32),
        pltpu.VMEM((_W,), jnp.int32),
        pltpu.VMEM((32,), jnp.int32),
        pltpu.VMEM((_NP // 16, 16), jnp.float32),
        pltpu.VMEM((5, 128), jnp.int32),
        pltpu.VMEM((40, 16), jnp.float32),
        pltpu.VMEM_SHARED((_NP // 16, 16), jnp.float32),
    ],
    compiler_params=_sc_params_nlp,
)
def _part_sc(row_hbm, col_hbm, ar_hbm, ac_hbm, br_hbm, bc_hbm, deg_hbm,
             rowf, colf, lar, lac, lbr, lbc, offs, degloc, idxv, zdeg, dacc):
    cid = lax.axis_index("c")
    sid = lax.axis_index("s")
    wid = cid * _NS + sid

    pltpu.sync_copy(row_hbm.at[wid], rowf)
    pltpu.sync_copy(col_hbm.at[wid], colf)

    lane = lax.iota(jnp.int32, 16)
    zvec = jnp.zeros((16,), jnp.float32)

    def zdl(i, carry):
        degloc[i, pl.ds(0, 16)] = zvec
        return carry

    lax.fori_loop(0, _NP // 16, zdl, 0)
    _fill_vmem_2d(zdeg, 40, 16, zvec)

    def fidx(i, carry):
        for q in range(8):
            idxv[i, pl.ds(q * 16, 16)] = i * 128 + q * 16 + lane
        return carry

    lax.fori_loop(0, 5, fidx, 0)
    pltpu.sync_copy(zdeg, dacc.at[pl.ds(sid * 40, 40)])
    plsc.subcore_barrier()

    def prefill(i, carry):
        spread = i * 16 + lane
        lar[pl.ds(i * 16, 16)] = spread & 1023
        lac[pl.ds(i * 16, 16)] = _NH + (spread & 511)
        lbr[pl.ds(i * 16, 16)] = spread & 1023
        lbc[pl.ds(i * 16, 16)] = _NH + (spread & 511)
        return carry

    lax.fori_loop(0, _W // 16, prefill, 0)

    offs[pl.ds(0, 16)] = jnp.zeros((16,), jnp.int32)
    offs[pl.ds(16, 16)] = jnp.zeros((16,), jnp.int32)
    lane1 = lax.iota(jnp.int32, 16) + 1

    ones16 = jnp.ones((16,), jnp.float32)

    def halfstep(base, offa, offb):
        rv = rowf[pl.ds(base, 16)]
        cv = colf[pl.ds(base, 16)]
        ma = cv < _NH
        mb = jnp.logical_not(ma)
        mai = jnp.where(ma, 1, 0)
        plsc.addupdate_scatter(
            degloc, [lax.shift_right_logical(cv, 4), cv & 15], ones16
        )
        incla = plsc.cumsum(mai)
        plsc.store_scatter(lar, [offa + incla - mai], rv, mask=ma)
        plsc.store_scatter(lac, [offa + incla - mai], cv, mask=ma)
        exclb = (lane1 - incla) - jnp.where(mb, 1, 0)
        plsc.store_scatter(lbr, [offb + exclb], rv, mask=mb)
        plsc.store_scatter(lbc, [offb + exclb], cv - _NH, mask=mb)
        tota = plsc.cummax(lax.rev(incla, (0,)))
        return tota

    def step(k, carry):
        offa = offs[pl.ds(0, 16)]
        offb = offs[pl.ds(16, 16)]
        tota1 = halfstep(k * 32, offa, offb)
        tota2 = halfstep(k * 32 + 16, offa + tota1, offb + 16 - tota1)
        offs[pl.ds(0, 16)] = jnp.minimum(offa + tota1 + tota2, _W - 32)
        offs[pl.ds(16, 16)] = jnp.minimum(offb + 32 - tota1 - tota2, _W - 32)
        return carry

    lax.fori_loop(0, _EPW // 32, step, 0)
    offa = offs[pl.ds(0, 16)]
    offb = offs[pl.ds(16, 16)]
    halfstep(_EPW - 16, offa, offb)

    pltpu.sync_copy(lar, ar_hbm.at[wid])
    pltpu.sync_copy(lac, ac_hbm.at[wid])
    pltpu.sync_copy(lbr, br_hbm.at[wid])
    pltpu.sync_copy(lbc, bc_hbm.at[wid])

    for j in range(5):
        pltpu.sync_copy(
            degloc.at[pl.ds(j * 128, 128)], dacc.at[idxv.at[j]], add=True
        )
    plsc.subcore_barrier()
    pltpu.sync_copy(
        dacc.at[pl.ds(sid * 40, 40)], deg_hbm.at[cid, pl.ds(sid * 40, 40)]
    )


@functools.partial(
    pl.kernel,
    mesh=_mesh,
    out_type=jax.ShapeDtypeStruct((_NP, _DOUT), jnp.float32),
    scratch_types=[
        pltpu.VMEM((_CHP, _C), jnp.int32),
        pltpu.VMEM((_CHP, _C), jnp.int32),
        pltpu.VMEM((_C, _DOUT), jnp.float32),
        pltpu.VMEM((_C, _DOUT), jnp.float32),
        pltpu.VMEM((_C, _DOUT), jnp.float32),
        pltpu.VMEM((_C, _DOUT), jnp.float32),
        pltpu.VMEM((_CSTR, _DOUT), jnp.float32),
        pltpu.VMEM((_CSTR, _DOUT), jnp.float32),
        pltpu.VMEM_SHARED((_ACCR, _DOUT), jnp.float32),
        [pltpu.SemaphoreType.DMA] * 4,
        [pltpu.SemaphoreType.DMA] * 4,
        pltpu.SemaphoreType.DMA,
        pltpu.SemaphoreType.DMA,
    ],
    compiler_params=_sc_params,
)
def _hop_sc(g_hbm, row_hbm, col_hbm, d2_hbm, gout_hbm, rowv, colv,
            b0, b1, b2, b3, ab, db, acc, gsem, ssem, zsem, dsem):
    cid = lax.axis_index("c")
    sid = lax.axis_index("s")
    bufs = (b0, b1, b2, b3)

    def g_start(j, slot):
        pltpu.async_copy(g_hbm.at[rowv.at[j]], bufs[slot], gsem[slot])

    def g_wait(slot):
        pltpu.make_async_copy(g_hbm.at[rowv.at[0]], bufs[slot], gsem[slot]).wait()

    def s_start(j, slot):
        pltpu.async_copy(bufs[slot], acc.at[colv.at[j]], ssem[slot], add=True)

    def s_wait(slot):
        pltpu.make_async_copy(bufs[slot], acc.at[colv.at[0]], ssem[slot]).wait()

    pltpu.async_copy(
        g_hbm.at[pl.ds(cid * _NH + sid * _CSTR, _CSTR)],
        acc.at[pl.ds(sid * _CSTR, _CSTR)],
        zsem,
    )
    pltpu.async_copy(
        d2_hbm.at[pl.ds(cid * _NH + sid * _CSTR, _CSTR)], db, dsem
    )
    pltpu.async_copy(row_hbm.at[cid, sid], rowv, zsem)
    pltpu.async_copy(col_hbm.at[cid, sid], colv, zsem)
    pltpu.make_async_copy(row_hbm.at[cid, sid], rowv, zsem).wait()
    pltpu.make_async_copy(col_hbm.at[cid, sid], colv, zsem).wait()

    g_start(0, 0)
    g_start(1, 1)
    pltpu.make_async_copy(
        g_hbm.at[pl.ds(cid * _NH, _CSTR)],
        acc.at[pl.ds(sid * _CSTR, _CSTR)],
        zsem,
    ).wait()
    plsc.subcore_barrier()
    g_wait(0); s_start(0, 0); g_start(2, 2)
    g_wait(1); s_start(1, 1); g_start(3, 3)
    g_wait(2); s_start(2, 2); s_wait(0); g_start(4, 0)
    g_wait(3); s_start(3, 3); s_wait(1); g_start(5, 1)

    def group(gi, carry):
        base = 4 * gi
        for b in range(4):
            j = base + b
            g_wait(b)
            s_start(j, b)
            s_wait((b + 2) % 4)
            g_start(j + 2, (b + 2) % 4)
        return carry

    lax.fori_loop(1, _CH // 4, group, 0)
    g_wait(0)
    g_wait(1)
    s_wait(2)
    s_wait(3)

    plsc.subcore_barrier()

    pltpu.make_async_copy(
        d2_hbm.at[pl.ds(cid * _NH, _CSTR)], db, dsem
    ).wait()
    pltpu.sync_copy(acc.at[pl.ds(sid * _CSTR, _CSTR)], ab)

    def crow(i, carry):
        for q in range(_DOUT // 16):
            sl = pl.ds(q * 16, 16)
            ab[i, sl] = db[i, sl] * ab[i, sl]
        return carry

    lax.fori_loop(0, _CSTR, crow, 0)
    pltpu.sync_copy(ab, gout_hbm.at[pl.ds(cid * _NH + sid * _CSTR, _CSTR)])


def _mlp_body(x_ref, w1_ref, b1_ref, w2_ref, b2_ref, o_ref):
    h = jnp.dot(x_ref[...], w1_ref[...], preferred_element_type=jnp.float32)
    h = jnp.maximum(h + b1_ref[...], 0.0)
    o_ref[...] = (
        jnp.dot(h, w2_ref[...], preferred_element_type=jnp.float32) + b2_ref[...]
    )


_MLP_R = 1024


def _mlp(x, w1, b1, w2, b2):
    return pl.pallas_call(
        _mlp_body,
        grid=(_NP // _MLP_R,),
        in_specs=[
            pl.BlockSpec((_MLP_R, _DIN), lambda i: (i, 0)),
            pl.BlockSpec((_DIN, _DH), lambda i: (0, 0)),
            pl.BlockSpec((1, _DH), lambda i: (0, 0)),
            pl.BlockSpec((_DH, _DOUT), lambda i: (0, 0)),
            pl.BlockSpec((1, _DOUT), lambda i: (0, 0)),
        ],
        out_specs=pl.BlockSpec((_MLP_R, _DOUT), lambda i: (i, 0)),
        out_shape=jax.ShapeDtypeStruct((_NP, _DOUT), jnp.float32),
    )(x, w1, b1.reshape(1, _DH), w2, b2.reshape(1, _DOUT))


def _prep_body(dp_ref, h_ref, g_ref, d2_ref, sd_ref):
    deg = (dp_ref[0] + dp_ref[1] + 1.0)[:, None]
    r = deg.shape[0]
    dinv = lax.rsqrt(deg)
    dinvb = jnp.broadcast_to(dinv, (r, _DOUT))
    g_ref[...] = dinvb * h_ref[...]
    d2_ref[...] = dinvb * dinvb
    sd_ref[...] = jnp.broadcast_to(jnp.sqrt(deg), (r, _DOUT))


_EW_R = 1024


def _prep(dp, h0):
    return pl.pallas_call(
        _prep_body,
        grid=(_NP // _EW_R,),
        in_specs=[
            pl.BlockSpec((_NC, _EW_R), lambda i: (0, i)),
            pl.BlockSpec((_EW_R, _DOUT), lambda i: (i, 0)),
        ],
        out_specs=[
            pl.BlockSpec((_EW_R, _DOUT), lambda i: (i, 0)),
            pl.BlockSpec((_EW_R, _DOUT), lambda i: (i, 0)),
            pl.BlockSpec((_EW_R, _DOUT), lambda i: (i, 0)),
        ],
        out_shape=[
            jax.ShapeDtypeStruct((_NP, _DOUT), jnp.float32),
            jax.ShapeDtypeStruct((_NP, _DOUT), jnp.float32),
            jax.ShapeDtypeStruct((_NP, _DOUT), jnp.float32),
        ],
    )(dp, h0)


def _final_body(h_ref, sd_ref, t_ref, *refs):
    g_refs, hid_ref = refs[:_K], refs[_K]
    s = t_ref[0, 1] * g_refs[0][...]
    for k in range(1, _K):
        s = s + t_ref[0, k + 1] * g_refs[k][...]
    hid_ref[...] = t_ref[0, 0] * h_ref[...] + sd_ref[...] * s


def _final(h0, sd, tvec, gs):
    return pl.pallas_call(
        _final_body,
        grid=(_NP // _EW_R,),
        in_specs=[
            pl.BlockSpec((_EW_R, _DOUT), lambda i: (i, 0)),
            pl.BlockSpec((_EW_R, _DOUT), lambda i: (i, 0)),
            pl.BlockSpec((1, _K + 1), lambda i: (0, 0)),
        ] + [pl.BlockSpec((_EW_R, _DOUT), lambda i: (i, 0)) for _ in range(_K)],
        out_specs=pl.BlockSpec((_EW_R, _DOUT), lambda i: (i, 0)),
        out_shape=jax.ShapeDtypeStruct((_NP, _DOUT), jnp.float32),
    )(h0, sd, tvec, *gs)


def kernel(x, edge_index, W1, b1, W2, b2, temp):
    row2 = edge_index[0].reshape(_NW, _EPW)
    col2 = edge_index[1].reshape(_NW, _EPW)

    xp = jnp.pad(x, ((0, _NP - _N), (0, 0)))
    h0 = _mlp(xp, W1, b1, W2, b2)
    ar, ac, br, bc, deg = _part_sc(row2, col2)
    g, d2, sd = _prep(deg.reshape(_NC, _NP), h0)

    def slabs(a, b):
        s = jnp.stack([a, b]).reshape(_NC, _NS, _CH, _C)
        return jnp.concatenate([s, s[:, :, :2]], axis=2)

    rows = slabs(ar, br)
    cols = slabs(ac, bc)

    gs = []
    for _ in range(_K):
        g = _hop_sc(g, rows, cols, d2)
        gs.append(g)

    hidden = _final(h0, sd, temp.reshape(1, _K + 1), gs)
    return hidden[:_N]


# speedup vs baseline: 8.6161x; 1.0004x over previous
import functools

import jax
import jax.numpy as jnp
from jax import lax
from jax.experimental import pallas as pl
from jax.experimental.pallas import tpu as pltpu
from jax.experimental.pallas import tpu_sc as plsc

_N = 10000
_NP = 10240
_NH = 5120
_E = 320000
_DIN = 128
_DH = 128
_DOUT = 64
_K = 10

_NC = 2
_NS = 16
_NW = _NC * _NS
_EPW = _E // _NW

_W = 5520
_ACCR = 5760
_CSTR = _NH // _NS
_C = 120
_CH = 2 * _W // _C
_CHP = _CH + 2

_mesh = plsc.VectorSubcoreMesh(core_axis_name="c", subcore_axis_name="s")
_sc_params = pltpu.CompilerParams(use_tc_tiling_on_sc=False)
_sc_params_nlp = pltpu.CompilerParams(use_tc_tiling_on_sc=False,
                                      needs_layout_passes=False)


def _fill_vmem_2d(ref, rows, cols, vec):
    def body(i, carry):
        for c in range(cols // 16):
            ref[i, pl.ds(c * 16, 16)] = vec
        return carry

    lax.fori_loop(0, rows, body, 0)


@functools.partial(
    pl.kernel,
    mesh=_mesh,
    out_type=[jax.ShapeDtypeStruct((_NW, _W), jnp.int32) for _ in range(4)]
    + [jax.ShapeDtypeStruct((_NC, _NP // 16, 16), jnp.float32)],
    scratch_types=[
        pltpu.VMEM((_EPW,), jnp.int32),
        pltpu.VMEM((_EPW,), jnp.int32),
        pltpu.VMEM((_W,), jnp.int32),
        pltpu.VMEM((_W,), jnp.int32),
        pltpu.VMEM((_W,), jnp.int32),
        pltpu.VMEM((_W,), jnp.int32),
        pltpu.VMEM((32,), jnp.int32),
        pltpu.VMEM((_NP // 16, 16), jnp.float32),
        pltpu.VMEM((5, 128), jnp.int32),
        pltpu.VMEM((40, 16), jnp.float32),
        pltpu.VMEM_SHARED((_NP // 16, 16), jnp.float32),
    ],
    compiler_params=_sc_params_nlp,
)
def _part_sc(row_hbm, col_hbm, ar_hbm, ac_hbm, br_hbm, bc_hbm, deg_hbm,
             rowf, colf, lar, lac, lbr, lbc, offs, degloc, idxv, zdeg, dacc):
    cid = lax.axis_index("c")
    sid = lax.axis_index("s")
    wid = cid * _NS + sid

    pltpu.sync_copy(row_hbm.at[wid], rowf)
    pltpu.sync_copy(col_hbm.at[wid], colf)

    lane = lax.iota(jnp.int32, 16)
    zvec = jnp.zeros((16,), jnp.float32)

    def zdl(i, carry):
        degloc[i, pl.ds(0, 16)] = zvec
        return carry

    lax.fori_loop(0, _NP // 16, zdl, 0)
    _fill_vmem_2d(zdeg, 40, 16, zvec)

    def fidx(i, carry):
        for q in range(8):
            idxv[i, pl.ds(q * 16, 16)] = i * 128 + q * 16 + lane
        return carry

    lax.fori_loop(0, 5, fidx, 0)
    pltpu.sync_copy(zdeg, dacc.at[pl.ds(sid * 40, 40)])
    plsc.subcore_barrier()

    def prefill(i, carry):
        spread = i * 16 + lane
        lar[pl.ds(i * 16, 16)] = spread & 1023
        lac[pl.ds(i * 16, 16)] = _NH + (spread & 511)
        lbr[pl.ds(i * 16, 16)] = spread & 1023
        lbc[pl.ds(i * 16, 16)] = _NH + (spread & 511)
        return carry

    lax.fori_loop(0, _W // 16, prefill, 0)

    offs[pl.ds(0, 16)] = jnp.zeros((16,), jnp.int32)
    offs[pl.ds(16, 16)] = jnp.zeros((16,), jnp.int32)
    lane1 = lax.iota(jnp.int32, 16) + 1

    ones16 = jnp.ones((16,), jnp.float32)

    def halfstep(base, offa, offb):
        rv = rowf[pl.ds(base, 16)]
        cv = colf[pl.ds(base, 16)]
        ma = cv < _NH
        mb = jnp.logical_not(ma)
        mai = jnp.where(ma, 1, 0)
        plsc.addupdate_scatter(
            degloc, [lax.shift_right_logical(cv, 4), cv & 15], ones16
        )
        incla = plsc.cumsum(mai)
        plsc.store_scatter(lar, [offa + incla - mai], rv, mask=ma)
        plsc.store_scatter(lac, [offa + incla - mai], cv, mask=ma)
        exclb = (lane1 - incla) - jnp.where(mb, 1, 0)
        plsc.store_scatter(lbr, [offb + exclb], rv, mask=mb)
        plsc.store_scatter(lbc, [offb + exclb], cv - _NH, mask=mb)
        tota = plsc.cummax(lax.rev(incla, (0,)))
        return tota

    def step(k, carry):
        offa = offs[pl.ds(0, 16)]
        offb = offs[pl.ds(16, 16)]
        tota1 = halfstep(k * 32, offa, offb)
        tota2 = halfstep(k * 32 + 16, offa + tota1, offb + 16 - tota1)
        offs[pl.ds(0, 16)] = jnp.minimum(offa + tota1 + tota2, _W - 32)
        offs[pl.ds(16, 16)] = jnp.minimum(offb + 32 - tota1 - tota2, _W - 32)
        return carry

    lax.fori_loop(0, _EPW // 32, step, 0)
    offa = offs[pl.ds(0, 16)]
    offb = offs[pl.ds(16, 16)]
    halfstep(_EPW - 16, offa, offb)

    pltpu.sync_copy(lar, ar_hbm.at[wid])
    pltpu.sync_copy(lac, ac_hbm.at[wid])
    pltpu.sync_copy(lbr, br_hbm.at[wid])
    pltpu.sync_copy(lbc, bc_hbm.at[wid])

    for j in range(5):
        pltpu.sync_copy(
            degloc.at[pl.ds(j * 128, 128)], dacc.at[idxv.at[j]], add=True
        )
    plsc.subcore_barrier()
    pltpu.sync_copy(
        dacc.at[pl.ds(sid * 40, 40)], deg_hbm.at[cid, pl.ds(sid * 40, 40)]
    )


@functools.partial(
    pl.kernel,
    mesh=_mesh,
    out_type=jax.ShapeDtypeStruct((_NP, _DOUT), jnp.float32),
    scratch_types=[
        pltpu.VMEM((_CHP, _C), jnp.int32),
        pltpu.VMEM((_CHP, _C), jnp.int32),
        pltpu.VMEM((_C, _DOUT), jnp.float32),
        pltpu.VMEM((_C, _DOUT), jnp.float32),
        pltpu.VMEM((_C, _DOUT), jnp.float32),
        pltpu.VMEM((_C, _DOUT), jnp.float32),
        pltpu.VMEM((_CSTR, _DOUT), jnp.float32),
        pltpu.VMEM((_CSTR, _DOUT), jnp.float32),
        pltpu.VMEM_SHARED((_ACCR, _DOUT), jnp.float32),
        [pltpu.SemaphoreType.DMA] * 4,
        [pltpu.SemaphoreType.DMA] * 4,
        pltpu.SemaphoreType.DMA,
        pltpu.SemaphoreType.DMA,
    ],
    compiler_params=_sc_params,
)
def _hop_sc(g_hbm, row_hbm, col_hbm, d2_hbm, gout_hbm, rowv, colv,
            b0, b1, b2, b3, ab, db, acc, gsem, ssem, zsem, dsem):
    cid = lax.axis_index("c")
    sid = lax.axis_index("s")
    bufs = (b0, b1, b2, b3)

    def g_start(j, slot):
        pltpu.async_copy(g_hbm.at[rowv.at[j]], bufs[slot], gsem[slot])

    def g_wait(slot):
        pltpu.make_async_copy(g_hbm.at[rowv.at[0]], bufs[slot], gsem[slot]).wait()

    def s_start(j, slot):
        pltpu.async_copy(bufs[slot], acc.at[colv.at[j]], ssem[slot], add=True)

    def s_wait(slot):
        pltpu.make_async_copy(bufs[slot], acc.at[colv.at[0]], ssem[slot]).wait()

    pltpu.async_copy(
        g_hbm.at[pl.ds(cid * _NH + sid * _CSTR, _CSTR)],
        acc.at[pl.ds(sid * _CSTR, _CSTR)],
        zsem,
    )
    pltpu.async_copy(
        d2_hbm.at[pl.ds(cid * _NH + sid * _CSTR, _CSTR)], db, dsem
    )
    pltpu.async_copy(row_hbm.at[cid, sid], rowv, zsem)
    pltpu.async_copy(col_hbm.at[cid, sid], colv, zsem)
    pltpu.make_async_copy(row_hbm.at[cid, sid], rowv, zsem).wait()
    pltpu.make_async_copy(col_hbm.at[cid, sid], colv, zsem).wait()

    g_start(0, 0)
    g_start(1, 1)
    pltpu.make_async_copy(
        g_hbm.at[pl.ds(cid * _NH, _CSTR)],
        acc.at[pl.ds(sid * _CSTR, _CSTR)],
        zsem,
    ).wait()
    plsc.subcore_barrier()
    g_wait(0); s_start(0, 0); g_start(2, 2)
    g_wait(1); s_start(1, 1); g_start(3, 3)
    g_wait(2); s_start(2, 2); s_wait(0); g_start(4, 0)
    g_wait(3); s_start(3, 3); s_wait(1); g_start(5, 1)

    def group(gi, carry):
        base = 4 * gi
        for b in range(4):
            j = base + b
            g_wait(b)
            s_start(j, b)
            s_wait((b + 2) % 4)
            g_start(j + 2, (b + 2) % 4)
        return carry

    lax.fori_loop(1, _CH // 4, group, 0)
    g_wait(0)
    g_wait(1)
    s_wait(2)
    s_wait(3)

    plsc.subcore_barrier()

    pltpu.make_async_copy(
        d2_hbm.at[pl.ds(cid * _NH, _CSTR)], db, dsem
    ).wait()
    pltpu.sync_copy(acc.at[pl.ds(sid * _CSTR, _CSTR)], ab)

    def crow(i, carry):
        for q in range(_DOUT // 16):
            sl = pl.ds(q * 16, 16)
            ab[i, sl] = db[i, sl] * ab[i, sl]
        return carry

    lax.fori_loop(0, _CSTR, crow, 0)
    pltpu.sync_copy(ab, gout_hbm.at[pl.ds(cid * _NH + sid * _CSTR, _CSTR)])


def _mlp_body(x_ref, w1_ref, b1_ref, w2_ref, b2_ref, o_ref):
    h = jnp.dot(x_ref[...], w1_ref[...], preferred_element_type=jnp.float32)
    h = jnp.maximum(h + b1_ref[...], 0.0)
    o_ref[...] = (
        jnp.dot(h, w2_ref[...], preferred_element_type=jnp.float32) + b2_ref[...]
    )


_MLP_R = 1024


def _mlp(x, w1, b1, w2, b2):
    return pl.pallas_call(
        _mlp_body,
        grid=(_NP // _MLP_R,),
        in_specs=[
            pl.BlockSpec((_MLP_R, _DIN), lambda i: (i, 0)),
            pl.BlockSpec((_DIN, _DH), lambda i: (0, 0)),
            pl.BlockSpec((1, _DH), lambda i: (0, 0)),
            pl.BlockSpec((_DH, _DOUT), lambda i: (0, 0)),
            pl.BlockSpec((1, _DOUT), lambda i: (0, 0)),
        ],
        out_specs=pl.BlockSpec((_MLP_R, _DOUT), lambda i: (i, 0)),
        out_shape=jax.ShapeDtypeStruct((_NP, _DOUT), jnp.float32),
    )(x, w1, b1.reshape(1, _DH), w2, b2.reshape(1, _DOUT))


def _prep_body(dp_ref, h_ref, g_ref, d2_ref, sd_ref):
    deg = (dp_ref[0] + dp_ref[1] + 1.0)[:, None]
    r = deg.shape[0]
    dinv = lax.rsqrt(deg)
    dinvb = jnp.broadcast_to(dinv, (r, _DOUT))
    g_ref[...] = dinvb * h_ref[...]
    d2_ref[...] = dinvb * dinvb
    sd_ref[...] = jnp.broadcast_to(jnp.sqrt(deg), (r, _DOUT))


_EW_R = 1024


def _prep(dp, h0):
    return pl.pallas_call(
        _prep_body,
        grid=(_NP // _EW_R,),
        in_specs=[
            pl.BlockSpec((_NC, _EW_R), lambda i: (0, i)),
            pl.BlockSpec((_EW_R, _DOUT), lambda i: (i, 0)),
        ],
        out_specs=[
            pl.BlockSpec((_EW_R, _DOUT), lambda i: (i, 0)),
            pl.BlockSpec((_EW_R, _DOUT), lambda i: (i, 0)),
            pl.BlockSpec((_EW_R, _DOUT), lambda i: (i, 0)),
        ],
        out_shape=[
            jax.ShapeDtypeStruct((_NP, _DOUT), jnp.float32),
            jax.ShapeDtypeStruct((_NP, _DOUT), jnp.float32),
            jax.ShapeDtypeStruct((_NP, _DOUT), jnp.float32),
        ],
    )(dp, h0)


def _final_body(h_ref, sd_ref, t_ref, *refs):
    g_refs, hid_ref = refs[:_K], refs[_K]
    s = t_ref[0, 1] * g_refs[0][...]
    for k in range(1, _K):
        s = s + t_ref[0, k + 1] * g_refs[k][...]
    hid_ref[...] = t_ref[0, 0] * h_ref[...] + sd_ref[...] * s


def _final(h0, sd, tvec, gs):
    return pl.pallas_call(
        _final_body,
        grid=(_NP // _EW_R,),
        in_specs=[
            pl.BlockSpec((_EW_R, _DOUT), lambda i: (i, 0)),
            pl.BlockSpec((_EW_R, _DOUT), lambda i: (i, 0)),
            pl.BlockSpec((1, _K + 1), lambda i: (0, 0)),
        ] + [pl.BlockSpec((_EW_R, _DOUT), lambda i: (i, 0)) for _ in range(_K)],
        out_specs=pl.BlockSpec((_EW_R, _DOUT), lambda i: (i, 0)),
        out_shape=jax.ShapeDtypeStruct((_NP, _DOUT), jnp.float32),
    )(h0, sd, tvec, *gs)


def kernel(x, edge_index, W1, b1, W2, b2, temp):
    row2 = edge_index[0].reshape(_NW, _EPW)
    col2 = edge_index[1].reshape(_NW, _EPW)

    xp = jnp.pad(x, ((0, _NP - _N), (0, 0)))
    h0 = _mlp(xp, W1, b1, W2, b2)
    ar, ac, br, bc, deg = _part_sc(row2, col2)
    g, d2, sd = _prep(deg.reshape(_NC, _NP), h0)

    def slabs(a, b):
        s = jnp.stack([a, b]).reshape(_NC, _NS, _CH, _C)
        return jnp.concatenate([s, s[:, :, :2]], axis=2)

    rows = slabs(ar, br)
    cols = slabs(ac, bc)

    gs = []
    for _ in range(_K):
        g = _hop_sc(g, rows, cols, d2)
        gs.append(g)

    hidden = _final(h0, sd, temp.reshape(1, _K + 1), gs)
    return hidden[:_N]
